# Initial kernel scaffold; baseline (speedup 1.0000x reference)
import jax
import jax.numpy as jnp
from jax.experimental import pallas as pl


def kernel(x, edge_index, batch, W, att_src, att_dst, bias):
    raise NotImplementedError("write your pallas kernel here")


# speedup vs baseline: 8.7431x; 8.7431x over previous
import jax
import jax.numpy as jnp
from jax import lax
from jax.experimental import pallas as pl
from jax.experimental.pallas import tpu as pltpu
from jax.experimental.pallas import tpu_sc as plsc

N = 10000
E = 320000
F = 128
NPAD = 10240
SLICE = NPAD // 16
E_T = E // 16
ECH = 2000
ROWBLK = 16


def _pre_body(x_ref, w_ref, as_ref, ad_ref, h_ref, asrc_ref, adst_ref):
    h = jnp.dot(x_ref[...], w_ref[...], preferred_element_type=jnp.float32)
    h_ref[...] = h
    asrc_ref[...] = jnp.sum(h * as_ref[...], axis=1, keepdims=True)
    adst_ref[...] = jnp.sum(h * ad_ref[...], axis=1, keepdims=True)


def _tc_prologue(x, W, att_src, att_dst):
    blk = N // 10
    return pl.pallas_call(
        _pre_body,
        grid=(10,),
        in_specs=[
            pl.BlockSpec((blk, F), lambda i: (i, 0)),
            pl.BlockSpec((F, F), lambda i: (0, 0)),
            pl.BlockSpec((1, F), lambda i: (0, 0)),
            pl.BlockSpec((1, F), lambda i: (0, 0)),
        ],
        out_specs=[
            pl.BlockSpec((blk, F), lambda i: (i, 0)),
            pl.BlockSpec((blk, 1), lambda i: (i, 0)),
            pl.BlockSpec((blk, 1), lambda i: (i, 0)),
        ],
        out_shape=[
            jax.ShapeDtypeStruct((N, F), jnp.float32),
            jax.ShapeDtypeStruct((N, 1), jnp.float32),
            jax.ShapeDtypeStruct((N, 1), jnp.float32),
        ],
    )(x, W, att_src, att_dst)


def _leaky_exp(a):
    return jnp.exp(jnp.where(a > 0, a, 0.2 * a))


def _sc_body(asrc_hbm, adst_hbm, src_hbm, dst_hbm, h_hbm,
             alpha_e_hbm, alpha_self_hbm, dn_hbm, s_hbm,
             srcc_v, dstc_v, av_v, bv_v, dnv_v, alphac_v,
             rows_v, asl_v, bsl_v, sbuf_v,
             shd_s, shS_s, sem):
    sid = lax.axis_index("s")
    base = sid * SLICE
    e0 = sid * E_T

    zero16 = jnp.zeros((ROWBLK,), jnp.float32)

    def _zero_rows(i, _):
        rows_v[i // 8, pl.ds((i % 8) * 16, 16)] = zero16
        return 0
    lax.fori_loop(0, 16 * 8, _zero_rows, 0)
    def _zero_s(k, _):
        pltpu.sync_copy(rows_v, shS_s.at[pl.ds(base + k * ROWBLK, ROWBLK)])
        return 0
    lax.fori_loop(0, SLICE // ROWBLK, _zero_s, 0)

    def _zero_sbuf(i, _):
        sbuf_v[pl.ds(i * 16, 16)] = zero16
        return 0
    lax.fori_loop(0, SLICE // 16, _zero_sbuf, 0)
    pltpu.sync_copy(sbuf_v, shd_s.at[pl.ds(base, SLICE)])
    plsc.subcore_barrier()

    def _wcompute(i, _):
        a16 = av_v[pl.ds(i * 16, 16)] + bv_v[pl.ds(i * 16, 16)]
        av_v[pl.ds(i * 16, 16)] = _leaky_exp(a16)
        return 0

    for c in range(E_T // ECH):
        pltpu.sync_copy(src_hbm.at[pl.ds(e0 + c * ECH, ECH)], srcc_v)
        pltpu.sync_copy(dst_hbm.at[pl.ds(e0 + c * ECH, ECH)], dstc_v)
        pltpu.async_copy(asrc_hbm.at[srcc_v], av_v, sem).wait()
        pltpu.async_copy(adst_hbm.at[dstc_v], bv_v, sem).wait()
        lax.fori_loop(0, ECH // 16, _wcompute, 0)
        pltpu.sync_copy(av_v, shd_s.at[dstc_v], add=True)

    plsc.subcore_barrier()

    pltpu.sync_copy(asrc_hbm.at[pl.ds(base, SLICE)], asl_v)
    pltpu.sync_copy(adst_hbm.at[pl.ds(base, SLICE)], bsl_v)
    pltpu.sync_copy(shd_s.at[pl.ds(base, SLICE)], sbuf_v)
    def _selfloop(i, _):
        idx = pl.ds(i * 16, 16)
        wsel = _leaky_exp(asl_v[idx] + bsl_v[idx])
        tot = sbuf_v[idx] + wsel
        sbuf_v[idx] = tot
        asl_v[idx] = wsel / (tot + 1e-16)
        return 0
    lax.fori_loop(0, SLICE // 16, _selfloop, 0)
    pltpu.sync_copy(sbuf_v, dn_hbm.at[pl.ds(base, SLICE)])
    pltpu.sync_copy(asl_v, alpha_self_hbm.at[pl.ds(base, SLICE)])
    plsc.subcore_barrier()

    def _alpha(i, _):
        idx = pl.ds(i * 16, 16)
        w16 = _leaky_exp(av_v[idx] + bv_v[idx])
        alphac_v[idx] = w16 / (dnv_v[idx] + 1e-16)
        return 0

    def _rows(i, _):
        s16 = srcc_v[pl.ds(i * 16, 16)]
        d16 = dstc_v[pl.ds(i * 16, 16)]
        an = alphac_v[pl.ds(i * 16, 16)]
        pltpu.async_copy(h_hbm.at[s16], rows_v, sem).wait()
        for l in range(16):
            a_l = an[l]
            for j in range(8):
                rows_v[l, pl.ds(j * 16, 16)] = rows_v[l, pl.ds(j * 16, 16)] * a_l
        pltpu.sync_copy(rows_v, shS_s.at[d16], add=True)
        return 0

    for c in range(E_T // ECH):
        pltpu.sync_copy(src_hbm.at[pl.ds(e0 + c * ECH, ECH)], srcc_v)
        pltpu.sync_copy(dst_hbm.at[pl.ds(e0 + c * ECH, ECH)], dstc_v)
        pltpu.async_copy(asrc_hbm.at[srcc_v], av_v, sem).wait()
        pltpu.async_copy(adst_hbm.at[dstc_v], bv_v, sem).wait()
        pltpu.async_copy(dn_hbm.at[dstc_v], dnv_v, sem).wait()
        lax.fori_loop(0, ECH // 16, _alpha, 0)
        pltpu.sync_copy(alphac_v, alpha_e_hbm.at[pl.ds(e0 + c * ECH, ECH)])
        lax.fori_loop(0, ECH // ROWBLK, _rows, 0)

    plsc.subcore_barrier()
    pltpu.sync_copy(shS_s.at[pl.ds(base, SLICE)], s_hbm.at[pl.ds(base, SLICE)])


def _sc_edges(a_src, a_dst, src, dst, h):
    mesh = plsc.VectorSubcoreMesh(core_axis_name="c", subcore_axis_name="s",
                                  num_cores=1)
    kfn = pl.kernel(
        _sc_body,
        out_type=[
            jax.ShapeDtypeStruct((E,), jnp.float32),
            jax.ShapeDtypeStruct((NPAD,), jnp.float32),
            jax.ShapeDtypeStruct((NPAD,), jnp.float32),
            jax.ShapeDtypeStruct((NPAD, F), jnp.float32),
        ],
        mesh=mesh,
        compiler_params=pltpu.CompilerParams(needs_layout_passes=False),
        scratch_types=[
            pltpu.VMEM((ECH,), jnp.int32),
            pltpu.VMEM((ECH,), jnp.int32),
            pltpu.VMEM((ECH,), jnp.float32),
            pltpu.VMEM((ECH,), jnp.float32),
            pltpu.VMEM((ECH,), jnp.float32),
            pltpu.VMEM((ECH,), jnp.float32),
            pltpu.VMEM((ROWBLK, F), jnp.float32),
            pltpu.VMEM((SLICE,), jnp.float32),
            pltpu.VMEM((SLICE,), jnp.float32),
            pltpu.VMEM((SLICE,), jnp.float32),
            pltpu.VMEM_SHARED((NPAD,), jnp.float32),
            pltpu.VMEM_SHARED((NPAD, F), jnp.float32),
            pltpu.SemaphoreType.DMA,
        ],
    )
    return kfn(a_src, a_dst, src, dst, h)


def _post_body(s_ref, asel_ref, h_ref, b_ref, o_ref):
    v = s_ref[...] + asel_ref[...] * h_ref[...] + b_ref[...]
    o_ref[...] = jnp.where(v > 0, v, jnp.exp(jnp.minimum(v, 0.0)) - 1.0)


def _tc_epilogue(s_acc, alpha_self, h, bias):
    blk = N // 10
    return pl.pallas_call(
        _post_body,
        grid=(10,),
        in_specs=[
            pl.BlockSpec((blk, F), lambda i: (i, 0)),
            pl.BlockSpec((blk, 1), lambda i: (i, 0)),
            pl.BlockSpec((blk, F), lambda i: (i, 0)),
            pl.BlockSpec((1, F), lambda i: (0, 0)),
        ],
        out_specs=pl.BlockSpec((blk, F), lambda i: (i, 0)),
        out_shape=jax.ShapeDtypeStruct((N, F), jnp.float32),
    )(s_acc, alpha_self, h, bias)


@jax.jit
def kernel(x, edge_index, batch, W, att_src, att_dst, bias):
    src = edge_index[0]
    dst = edge_index[1]
    h, a_src2, a_dst2 = _tc_prologue(x, W, att_src.reshape(1, F), att_dst.reshape(1, F))
    a_src = jnp.pad(a_src2.reshape(N), (0, NPAD - N))
    a_dst = jnp.pad(a_dst2.reshape(N), (0, NPAD - N))

    alpha_e, alpha_self_pad, _dn, s_acc = _sc_edges(a_src, a_dst, src, dst, h)

    out = _tc_epilogue(s_acc[:N], alpha_self_pad[:N].reshape(N, 1), h,
                       bias.reshape(1, F))

    loop = jnp.arange(N, dtype=edge_index.dtype)
    ei = jnp.concatenate([edge_index, jnp.stack([loop, loop], axis=0)], axis=1)
    alpha = jnp.concatenate([alpha_e, alpha_self_pad[:N]]).reshape(E + N, 1)
    return out, ei, alpha


# speedup vs baseline: 19.1603x; 2.1915x over previous
import jax
import jax.numpy as jnp
from jax import lax
from jax.experimental import pallas as pl
from jax.experimental.pallas import tpu as pltpu
from jax.experimental.pallas import tpu_sc as plsc

N = 10000
E = 320000
F = 128
NPAD = 10240
SLICE = NPAD // 16
E_T = E // 16
ECH = 2000
ROWBLK = 16


def _pre_body(x_ref, w_ref, as_ref, ad_ref, h_ref, asrc_ref, adst_ref):
    h = jnp.dot(x_ref[...], w_ref[...], preferred_element_type=jnp.float32)
    h_ref[...] = h
    asrc_ref[...] = jnp.sum(h * as_ref[...], axis=1, keepdims=True)
    adst_ref[...] = jnp.sum(h * ad_ref[...], axis=1, keepdims=True)


def _tc_prologue(x, W, att_src, att_dst):
    blk = N // 10
    return pl.pallas_call(
        _pre_body,
        grid=(10,),
        in_specs=[
            pl.BlockSpec((blk, F), lambda i: (i, 0)),
            pl.BlockSpec((F, F), lambda i: (0, 0)),
            pl.BlockSpec((1, F), lambda i: (0, 0)),
            pl.BlockSpec((1, F), lambda i: (0, 0)),
        ],
        out_specs=[
            pl.BlockSpec((blk, F), lambda i: (i, 0)),
            pl.BlockSpec((blk, 1), lambda i: (i, 0)),
            pl.BlockSpec((blk, 1), lambda i: (i, 0)),
        ],
        out_shape=[
            jax.ShapeDtypeStruct((N, F), jnp.float32),
            jax.ShapeDtypeStruct((N, 1), jnp.float32),
            jax.ShapeDtypeStruct((N, 1), jnp.float32),
        ],
    )(x, W, att_src, att_dst)


def _leaky_exp(a):
    return jnp.exp(jnp.where(a > 0, a, 0.2 * a))


def _sc_body(asrc_hbm, adst_hbm, src_hbm, dst_hbm, h_hbm,
             alpha_e_hbm, alpha_self_hbm, dn_hbm, s_hbm,
             srcc_v, dstc_v, av_v, bv_v, dnv_v, alphac_v,
             rows_v, gb1_v, gb2_v, sb0_v, sb1_v, sb2_v,
             asl_v, bsl_v, sbuf_v,
             shd_s, shS_s, sem, gsem0, gsem1, gsem2, ssem0, ssem1, ssem2):
    sid = lax.axis_index("s")
    base = sid * SLICE
    e0 = sid * E_T

    zero16 = jnp.zeros((ROWBLK,), jnp.float32)

    def _zero_rows(i, _):
        rows_v[i // 8, pl.ds((i % 8) * 16, 16)] = zero16
        return 0
    lax.fori_loop(0, 16 * 8, _zero_rows, 0)
    def _zero_s(k, _):
        pltpu.sync_copy(rows_v, shS_s.at[pl.ds(base + k * ROWBLK, ROWBLK)])
        return 0
    lax.fori_loop(0, SLICE // ROWBLK, _zero_s, 0)

    def _zero_sbuf(i, _):
        sbuf_v[pl.ds(i * 16, 16)] = zero16
        return 0
    lax.fori_loop(0, SLICE // 16, _zero_sbuf, 0)
    pltpu.sync_copy(sbuf_v, shd_s.at[pl.ds(base, SLICE)])
    plsc.subcore_barrier()

    def _wcompute(i, _):
        a16 = av_v[pl.ds(i * 16, 16)] + bv_v[pl.ds(i * 16, 16)]
        av_v[pl.ds(i * 16, 16)] = _leaky_exp(a16)
        return 0

    for c in range(E_T // ECH):
        pltpu.sync_copy(src_hbm.at[pl.ds(e0 + c * ECH, ECH)], srcc_v)
        pltpu.sync_copy(dst_hbm.at[pl.ds(e0 + c * ECH, ECH)], dstc_v)
        pltpu.async_copy(asrc_hbm.at[srcc_v], av_v, sem).wait()
        pltpu.async_copy(adst_hbm.at[dstc_v], bv_v, sem).wait()
        lax.fori_loop(0, ECH // 16, _wcompute, 0)
        pltpu.sync_copy(av_v, shd_s.at[dstc_v], add=True)

    plsc.subcore_barrier()

    pltpu.sync_copy(asrc_hbm.at[pl.ds(base, SLICE)], asl_v)
    pltpu.sync_copy(adst_hbm.at[pl.ds(base, SLICE)], bsl_v)
    pltpu.sync_copy(shd_s.at[pl.ds(base, SLICE)], sbuf_v)
    def _selfloop(i, _):
        idx = pl.ds(i * 16, 16)
        wsel = _leaky_exp(asl_v[idx] + bsl_v[idx])
        tot = sbuf_v[idx] + wsel
        sbuf_v[idx] = tot
        asl_v[idx] = wsel / (tot + 1e-16)
        return 0
    lax.fori_loop(0, SLICE // 16, _selfloop, 0)
    pltpu.sync_copy(sbuf_v, dn_hbm.at[pl.ds(base, SLICE)])
    pltpu.sync_copy(asl_v, alpha_self_hbm.at[pl.ds(base, SLICE)])
    plsc.subcore_barrier()

    G = ECH // ROWBLK
    gbufs = (rows_v, gb1_v, gb2_v)
    sbufs = (sb0_v, sb1_v, sb2_v)
    gsems = (gsem0, gsem1, gsem2)
    ssems = (ssem0, ssem1, ssem2)

    def _alpha(i, _):
        idx = pl.ds(i * 16, 16)
        w16 = _leaky_exp(av_v[idx] + bv_v[idx])
        alphac_v[idx] = w16 / (dnv_v[idx] + 1e-16)
        return 0

    def _sidx(g):
        return srcc_v[pl.ds(g * 16, 16)]

    def _didx(g):
        return dstc_v[pl.ds(g * 16, 16)]

    def _start_g(g, b):
        pltpu.async_copy(h_hbm.at[_sidx(g)], gbufs[b], gsems[b])

    def _wait_g(g, b):
        pltpu.make_async_copy(h_hbm.at[_sidx(g)], gbufs[b], gsems[b]).wait()

    def _start_s(g, b):
        pltpu.async_copy(sbufs[b], shS_s.at[_didx(g)], ssems[b], add=True)

    def _wait_s(g, b):
        pltpu.make_async_copy(sbufs[b], shS_s.at[_didx(g)], ssems[b]).wait()

    def _scale(g, b):
        an = alphac_v[pl.ds(g * 16, 16)]
        for l in range(16):
            a_l = an[l]
            for j in range(8):
                sbufs[b][l, pl.ds(j * 16, 16)] = (
                    gbufs[b][l, pl.ds(j * 16, 16)] * a_l)
        return 0

    def _chunk(c, _):
        off = e0 + c * ECH
        pltpu.sync_copy(src_hbm.at[pl.ds(off, ECH)], srcc_v)
        pltpu.sync_copy(dst_hbm.at[pl.ds(off, ECH)], dstc_v)
        pltpu.async_copy(asrc_hbm.at[srcc_v], av_v, sem).wait()
        pltpu.async_copy(adst_hbm.at[dstc_v], bv_v, sem).wait()
        pltpu.async_copy(dn_hbm.at[dstc_v], dnv_v, sem).wait()
        lax.fori_loop(0, G, _alpha, 0)
        pltpu.sync_copy(alphac_v, alpha_e_hbm.at[pl.ds(off, ECH)])

        for b in range(3):
            _start_g(b, b)
        for b in range(3):
            _wait_g(b, b)
            _scale(b, b)
            _start_g(b + 3, b)
            _start_s(b, b)

        def _pipe(k, _):
            for b in range(3):
                g = 3 * k + b
                _wait_g(g, b)
                _wait_s(g - 3, b)
                _scale(g, b)
                @pl.when(g + 3 < G)
                def _():
                    _start_g(g + 3, b)
                _start_s(g, b)
            return 0
        lax.fori_loop(1, G // 3, _pipe, 0)

        for b, g in ((0, G - 2), (1, G - 1)):
            _wait_g(g, b)
            _wait_s(g - 3, b)
            _scale(g, b)
            _start_s(g, b)
        _wait_s(G - 3, 2)
        _wait_s(G - 2, 0)
        _wait_s(G - 1, 1)
        return 0

    lax.fori_loop(0, E_T // ECH, _chunk, 0)

    plsc.subcore_barrier()
    pltpu.sync_copy(shS_s.at[pl.ds(base, SLICE)], s_hbm.at[pl.ds(base, SLICE)])


def _sc_edges(a_src, a_dst, src, dst, h):
    mesh = plsc.VectorSubcoreMesh(core_axis_name="c", subcore_axis_name="s",
                                  num_cores=1)
    kfn = pl.kernel(
        _sc_body,
        out_type=[
            jax.ShapeDtypeStruct((E,), jnp.float32),
            jax.ShapeDtypeStruct((NPAD,), jnp.float32),
            jax.ShapeDtypeStruct((NPAD,), jnp.float32),
            jax.ShapeDtypeStruct((NPAD, F), jnp.float32),
        ],
        mesh=mesh,
        compiler_params=pltpu.CompilerParams(needs_layout_passes=False),
        scratch_types=[
            pltpu.VMEM((ECH,), jnp.int32),
            pltpu.VMEM((ECH,), jnp.int32),
            pltpu.VMEM((ECH,), jnp.float32),
            pltpu.VMEM((ECH,), jnp.float32),
            pltpu.VMEM((ECH,), jnp.float32),
            pltpu.VMEM((ECH,), jnp.float32),
            pltpu.VMEM((ROWBLK, F), jnp.float32),
            pltpu.VMEM((ROWBLK, F), jnp.float32),
            pltpu.VMEM((ROWBLK, F), jnp.float32),
            pltpu.VMEM((ROWBLK, F), jnp.float32),
            pltpu.VMEM((ROWBLK, F), jnp.float32),
            pltpu.VMEM((ROWBLK, F), jnp.float32),
            pltpu.VMEM((SLICE,), jnp.float32),
            pltpu.VMEM((SLICE,), jnp.float32),
            pltpu.VMEM((SLICE,), jnp.float32),
            pltpu.VMEM_SHARED((NPAD,), jnp.float32),
            pltpu.VMEM_SHARED((NPAD, F), jnp.float32),
            pltpu.SemaphoreType.DMA,
            pltpu.SemaphoreType.DMA,
            pltpu.SemaphoreType.DMA,
            pltpu.SemaphoreType.DMA,
            pltpu.SemaphoreType.DMA,
            pltpu.SemaphoreType.DMA,
            pltpu.SemaphoreType.DMA,
        ],
    )
    return kfn(a_src, a_dst, src, dst, h)


def _post_body(s_ref, asel_ref, h_ref, b_ref, o_ref):
    v = s_ref[...] + asel_ref[...] * h_ref[...] + b_ref[...]
    o_ref[...] = jnp.where(v > 0, v, jnp.exp(jnp.minimum(v, 0.0)) - 1.0)


def _tc_epilogue(s_acc, alpha_self, h, bias):
    blk = N // 10
    return pl.pallas_call(
        _post_body,
        grid=(10,),
        in_specs=[
            pl.BlockSpec((blk, F), lambda i: (i, 0)),
            pl.BlockSpec((blk, 1), lambda i: (i, 0)),
            pl.BlockSpec((blk, F), lambda i: (i, 0)),
            pl.BlockSpec((1, F), lambda i: (0, 0)),
        ],
        out_specs=pl.BlockSpec((blk, F), lambda i: (i, 0)),
        out_shape=jax.ShapeDtypeStruct((N, F), jnp.float32),
    )(s_acc, alpha_self, h, bias)


@jax.jit
def kernel(x, edge_index, batch, W, att_src, att_dst, bias):
    src = edge_index[0]
    dst = edge_index[1]
    h, a_src2, a_dst2 = _tc_prologue(x, W, att_src.reshape(1, F), att_dst.reshape(1, F))
    a_src = jnp.pad(a_src2.reshape(N), (0, NPAD - N))
    a_dst = jnp.pad(a_dst2.reshape(N), (0, NPAD - N))

    alpha_e, alpha_self_pad, _dn, s_acc = _sc_edges(a_src, a_dst, src, dst, h)

    out = _tc_epilogue(s_acc[:N], alpha_self_pad[:N].reshape(N, 1), h,
                       bias.reshape(1, F))

    loop = jnp.arange(N, dtype=edge_index.dtype)
    ei = jnp.concatenate([edge_index, jnp.stack([loop, loop], axis=0)], axis=1)
    alpha = jnp.concatenate([alpha_e, alpha_self_pad[:N]]).reshape(E + N, 1)
    return out, ei, alpha


# speedup vs baseline: 23.4037x; 1.2215x over previous
import jax
import jax.numpy as jnp
from jax import lax
from jax.experimental import pallas as pl
from jax.experimental.pallas import tpu as pltpu
from jax.experimental.pallas import tpu_sc as plsc

N = 10000
E = 320000
F = 128
FH = F // 2
NPAD = 10240
SLICE = NPAD // 16
E_T = E // 16
E_W = E // 32
ECH = 2000
ROWBLK = 16


def _pre_body(x_ref, w_ref, as_ref, ad_ref, h_ref, asrc_ref, adst_ref):
    h = jnp.dot(x_ref[...], w_ref[...], preferred_element_type=jnp.float32)
    h_ref[...] = h
    asrc_ref[...] = jnp.sum(h * as_ref[...], axis=1, keepdims=True)
    adst_ref[...] = jnp.sum(h * ad_ref[...], axis=1, keepdims=True)


def _tc_prologue(x, W, att_src, att_dst):
    blk = N // 10
    return pl.pallas_call(
        _pre_body,
        grid=(10,),
        in_specs=[
            pl.BlockSpec((blk, F), lambda i: (i, 0)),
            pl.BlockSpec((F, F), lambda i: (0, 0)),
            pl.BlockSpec((1, F), lambda i: (0, 0)),
            pl.BlockSpec((1, F), lambda i: (0, 0)),
        ],
        out_specs=[
            pl.BlockSpec((blk, F), lambda i: (i, 0)),
            pl.BlockSpec((blk, 1), lambda i: (i, 0)),
            pl.BlockSpec((blk, 1), lambda i: (i, 0)),
        ],
        out_shape=[
            jax.ShapeDtypeStruct((N, F), jnp.float32),
            jax.ShapeDtypeStruct((N, 1), jnp.float32),
            jax.ShapeDtypeStruct((N, 1), jnp.float32),
        ],
    )(x, W, att_src, att_dst)


def _leaky_exp(a):
    return jnp.exp(jnp.where(a > 0, a, 0.2 * a))


def _k2a_body(asrc_hbm, adst_hbm, src_hbm, dst_hbm,
              w_hbm, dpart_hbm,
              srcc_v, dstc_v, av_v, bv_v, sbuf_v,
              shd_s, sem):
    cid = lax.axis_index("c")
    sid = lax.axis_index("s")
    base = sid * SLICE
    e0 = (cid * 16 + sid) * E_W

    zero16 = jnp.zeros((16,), jnp.float32)

    def _zero_sbuf(i, _):
        sbuf_v[pl.ds(i * 16, 16)] = zero16
        return 0
    lax.fori_loop(0, SLICE // 16, _zero_sbuf, 0)
    pltpu.sync_copy(sbuf_v, shd_s.at[pl.ds(base, SLICE)])
    plsc.subcore_barrier()

    def _wcompute(i, _):
        a16 = av_v[pl.ds(i * 16, 16)] + bv_v[pl.ds(i * 16, 16)]
        av_v[pl.ds(i * 16, 16)] = _leaky_exp(a16)
        return 0

    def _chunk(c, _):
        off = e0 + c * ECH
        pltpu.sync_copy(src_hbm.at[pl.ds(off, ECH)], srcc_v)
        pltpu.sync_copy(dst_hbm.at[pl.ds(off, ECH)], dstc_v)
        pltpu.async_copy(asrc_hbm.at[srcc_v], av_v, sem).wait()
        pltpu.async_copy(adst_hbm.at[dstc_v], bv_v, sem).wait()
        lax.fori_loop(0, ECH // 16, _wcompute, 0)
        pltpu.sync_copy(av_v, w_hbm.at[pl.ds(off, ECH)])
        pltpu.sync_copy(av_v, shd_s.at[dstc_v], add=True)
        return 0
    lax.fori_loop(0, E_W // ECH, _chunk, 0)

    plsc.subcore_barrier()
    pltpu.sync_copy(shd_s.at[pl.ds(base, SLICE)],
                    dpart_hbm.at[pl.ds(cid * NPAD + base, SLICE)])


def _sc_denoms(a_src, a_dst, src, dst):
    mesh = plsc.VectorSubcoreMesh(core_axis_name="c", subcore_axis_name="s")
    kfn = pl.kernel(
        _k2a_body,
        out_type=[
            jax.ShapeDtypeStruct((E,), jnp.float32),
            jax.ShapeDtypeStruct((2 * NPAD,), jnp.float32),
        ],
        mesh=mesh,
        compiler_params=pltpu.CompilerParams(needs_layout_passes=False),
        scratch_types=[
            pltpu.VMEM((ECH,), jnp.int32),
            pltpu.VMEM((ECH,), jnp.int32),
            pltpu.VMEM((ECH,), jnp.float32),
            pltpu.VMEM((ECH,), jnp.float32),
            pltpu.VMEM((SLICE,), jnp.float32),
            pltpu.VMEM_SHARED((NPAD,), jnp.float32),
            pltpu.SemaphoreType.DMA,
        ],
    )
    return kfn(a_src, a_dst, src, dst)


def _k2b_body(asrc_hbm, adst_hbm, src_hbm, dst_hbm, w_hbm, dpart_hbm, h2_hbm,
              alpha_e_hbm, alpha_self_hbm, dntot_hbm, s2_hbm,
              srcc_v, dstc_v, wv_v, dnv_v,
              gb0_v, gb1_v, gb2_v, sb0_v, sb1_v, sb2_v,
              asl_v, bsl_v, d0_v, d1_v,
              shS_s, sem, gsem0, gsem1, gsem2, ssem0, ssem1, ssem2):
    cid = lax.axis_index("c")
    sid = lax.axis_index("s")
    base = sid * SLICE
    e0 = sid * E_T
    crow = cid * N

    zero16 = jnp.zeros((16,), jnp.float32)
    G = ECH // ROWBLK
    gbufs = (gb0_v, gb1_v, gb2_v)
    sbufs = (sb0_v, sb1_v, sb2_v)
    gsems = (gsem0, gsem1, gsem2)
    ssems = (ssem0, ssem1, ssem2)

    def _zero_rows(i, _):
        gb0_v[i // 4, pl.ds((i % 4) * 16, 16)] = zero16
        return 0
    lax.fori_loop(0, 16 * 4, _zero_rows, 0)
    def _zero_s(k, _):
        pltpu.sync_copy(gb0_v, shS_s.at[pl.ds(base + k * ROWBLK, ROWBLK)])
        return 0
    lax.fori_loop(0, SLICE // ROWBLK, _zero_s, 0)

    pltpu.sync_copy(asrc_hbm.at[pl.ds(base, SLICE)], asl_v)
    pltpu.sync_copy(adst_hbm.at[pl.ds(base, SLICE)], bsl_v)
    pltpu.sync_copy(dpart_hbm.at[pl.ds(base, SLICE)], d0_v)
    pltpu.sync_copy(dpart_hbm.at[pl.ds(NPAD + base, SLICE)], d1_v)
    def _selfloop(i, _):
        idx = pl.ds(i * 16, 16)
        wsel = _leaky_exp(asl_v[idx] + bsl_v[idx])
        tot = d0_v[idx] + d1_v[idx] + wsel
        d0_v[idx] = tot
        asl_v[idx] = wsel / (tot + 1e-16)
        return 0
    lax.fori_loop(0, SLICE // 16, _selfloop, 0)
    pltpu.sync_copy(d0_v, dntot_hbm.at[pl.ds(base, SLICE)])
    @pl.when(cid == 0)
    def _():
        pltpu.sync_copy(asl_v, alpha_self_hbm.at[pl.ds(base, SLICE)])
    plsc.subcore_barrier()

    def _alpha(i, _):
        idx = pl.ds(i * 16, 16)
        wv_v[idx] = wv_v[idx] / (dnv_v[idx] + 1e-16)
        return 0

    def _sidx(g):
        return srcc_v[pl.ds(g * 16, 16)] + crow

    def _didx(g):
        return dstc_v[pl.ds(g * 16, 16)]

    def _start_g(g, b):
        pltpu.async_copy(h2_hbm.at[_sidx(g)], gbufs[b], gsems[b])

    def _wait_g(g, b):
        pltpu.make_async_copy(h2_hbm.at[_sidx(g)], gbufs[b], gsems[b]).wait()

    def _start_s(g, b):
        pltpu.async_copy(sbufs[b], shS_s.at[_didx(g)], ssems[b], add=True)

    def _wait_s(g, b):
        pltpu.make_async_copy(sbufs[b], shS_s.at[_didx(g)], ssems[b]).wait()

    def _scale(g, b):
        an = wv_v[pl.ds(g * 16, 16)]
        for l in range(16):
            a_l = an[l]
            for j in range(4):
                sbufs[b][l, pl.ds(j * 16, 16)] = (
                    gbufs[b][l, pl.ds(j * 16, 16)] * a_l)
        return 0

    def _chunk(c, _):
        off = e0 + c * ECH
        pltpu.sync_copy(src_hbm.at[pl.ds(off, ECH)], srcc_v)
        pltpu.sync_copy(dst_hbm.at[pl.ds(off, ECH)], dstc_v)
        pltpu.sync_copy(w_hbm.at[pl.ds(off, ECH)], wv_v)
        pltpu.async_copy(dntot_hbm.at[dstc_v], dnv_v, sem).wait()
        lax.fori_loop(0, G, _alpha, 0)
        @pl.when(cid == 0)
        def _():
            pltpu.sync_copy(wv_v, alpha_e_hbm.at[pl.ds(off, ECH)])

        for b in range(3):
            _start_g(b, b)
        for b in range(3):
            _wait_g(b, b)
            _scale(b, b)
            _start_g(b + 3, b)
            _start_s(b, b)

        def _pipe(k, _):
            for b in range(3):
                g = 3 * k + b
                _wait_g(g, b)
                _wait_s(g - 3, b)
                _scale(g, b)
                @pl.when(g + 3 < G)
                def _():
                    _start_g(g + 3, b)
                _start_s(g, b)
            return 0
        lax.fori_loop(1, G // 3, _pipe, 0)

        for b, g in ((0, G - 2), (1, G - 1)):
            _wait_g(g, b)
            _wait_s(g - 3, b)
            _scale(g, b)
            _start_s(g, b)
        _wait_s(G - 3, 2)
        _wait_s(G - 2, 0)
        _wait_s(G - 1, 1)
        return 0

    lax.fori_loop(0, E_T // ECH, _chunk, 0)

    plsc.subcore_barrier()
    pltpu.sync_copy(shS_s.at[pl.ds(base, SLICE)],
                    s2_hbm.at[pl.ds(cid * NPAD + base, SLICE)])


def _sc_rows(a_src, a_dst, src, dst, w_e, dpart, h2):
    mesh = plsc.VectorSubcoreMesh(core_axis_name="c", subcore_axis_name="s")
    kfn = pl.kernel(
        _k2b_body,
        out_type=[
            jax.ShapeDtypeStruct((E,), jnp.float32),
            jax.ShapeDtypeStruct((NPAD,), jnp.float32),
            jax.ShapeDtypeStruct((NPAD,), jnp.float32),
            jax.ShapeDtypeStruct((2 * NPAD, FH), jnp.float32),
        ],
        mesh=mesh,
        compiler_params=pltpu.CompilerParams(needs_layout_passes=False,
                                             use_tc_tiling_on_sc=False),
        scratch_types=[
            pltpu.VMEM((ECH,), jnp.int32),
            pltpu.VMEM((ECH,), jnp.int32),
            pltpu.VMEM((ECH,), jnp.float32),
            pltpu.VMEM((ECH,), jnp.float32),
            pltpu.VMEM((ROWBLK, FH), jnp.float32),
            pltpu.VMEM((ROWBLK, FH), jnp.float32),
            pltpu.VMEM((ROWBLK, FH), jnp.float32),
            pltpu.VMEM((ROWBLK, FH), jnp.float32),
            pltpu.VMEM((ROWBLK, FH), jnp.float32),
            pltpu.VMEM((ROWBLK, FH), jnp.float32),
            pltpu.VMEM((SLICE,), jnp.float32),
            pltpu.VMEM((SLICE,), jnp.float32),
            pltpu.VMEM((SLICE,), jnp.float32),
            pltpu.VMEM((SLICE,), jnp.float32),
            pltpu.VMEM_SHARED((NPAD, FH), jnp.float32),
            pltpu.SemaphoreType.DMA,
            pltpu.SemaphoreType.DMA,
            pltpu.SemaphoreType.DMA,
            pltpu.SemaphoreType.DMA,
            pltpu.SemaphoreType.DMA,
            pltpu.SemaphoreType.DMA,
            pltpu.SemaphoreType.DMA,
        ],
    )
    return kfn(a_src, a_dst, src, dst, w_e, dpart, h2)


def _post_body(s_ref, asel_ref, h_ref, b_ref, o_ref):
    s = jnp.concatenate([s_ref[0], s_ref[1]], axis=-1)
    v = s + asel_ref[...] * h_ref[...] + b_ref[...]
    o_ref[...] = jnp.where(v > 0, v, jnp.exp(jnp.minimum(v, 0.0)) - 1.0)


def _tc_epilogue(s2, alpha_self, h, bias):
    blk = N // 10
    return pl.pallas_call(
        _post_body,
        grid=(10,),
        in_specs=[
            pl.BlockSpec((2, blk, FH), lambda i: (0, i, 0)),
            pl.BlockSpec((blk, 1), lambda i: (i, 0)),
            pl.BlockSpec((blk, F), lambda i: (i, 0)),
            pl.BlockSpec((1, F), lambda i: (0, 0)),
        ],
        out_specs=pl.BlockSpec((blk, F), lambda i: (i, 0)),
        out_shape=jax.ShapeDtypeStruct((N, F), jnp.float32),
    )(s2, alpha_self, h, bias)


@jax.jit
def kernel(x, edge_index, batch, W, att_src, att_dst, bias):
    src = edge_index[0]
    dst = edge_index[1]
    h, a_src2, a_dst2 = _tc_prologue(x, W, att_src.reshape(1, F), att_dst.reshape(1, F))
    a_src = jnp.pad(a_src2.reshape(N), (0, NPAD - N))
    a_dst = jnp.pad(a_dst2.reshape(N), (0, NPAD - N))

    w_e, dpart = _sc_denoms(a_src, a_dst, src, dst)

    h2 = jnp.transpose(h.reshape(N, 2, FH), (1, 0, 2)).reshape(2 * N, FH)
    alpha_e, alpha_self_pad, _dn, s2 = _sc_rows(
        a_src, a_dst, src, dst, w_e, dpart, h2)

    s_halves = s2.reshape(2, NPAD, FH)[:, :N, :]
    out = _tc_epilogue(s_halves, alpha_self_pad[:N].reshape(N, 1), h,
                       bias.reshape(1, F))

    loop = jnp.arange(N, dtype=edge_index.dtype)
    ei = jnp.concatenate([edge_index, jnp.stack([loop, loop], axis=0)], axis=1)
    alpha = jnp.concatenate([alpha_e, alpha_self_pad[:N]]).reshape(E + N, 1)
    return out, ei, alpha


# speedup vs baseline: 28.1433x; 1.2025x over previous
import jax
import jax.numpy as jnp
from jax import lax
from jax.experimental import pallas as pl
from jax.experimental.pallas import tpu as pltpu
from jax.experimental.pallas import tpu_sc as plsc

N = 10000
E = 320000
F = 128
FH = F // 2
NPAD = 10240
SLICE = NPAD // 16
E_T = E // 16
E_W = E // 32
ECH = 2000
ECHB = 4000
ROWBLK = 32


def _pre_body(x_ref, w_ref, as_ref, ad_ref, h_ref, asrc_ref, adst_ref):
    h = jnp.dot(x_ref[...], w_ref[...], preferred_element_type=jnp.float32)
    h_ref[...] = h
    asrc_ref[...] = jnp.sum(h * as_ref[...], axis=1, keepdims=True)
    adst_ref[...] = jnp.sum(h * ad_ref[...], axis=1, keepdims=True)


def _tc_prologue(x, W, att_src, att_dst):
    blk = N // 10
    return pl.pallas_call(
        _pre_body,
        grid=(10,),
        in_specs=[
            pl.BlockSpec((blk, F), lambda i: (i, 0)),
            pl.BlockSpec((F, F), lambda i: (0, 0)),
            pl.BlockSpec((1, F), lambda i: (0, 0)),
            pl.BlockSpec((1, F), lambda i: (0, 0)),
        ],
        out_specs=[
            pl.BlockSpec((blk, F), lambda i: (i, 0)),
            pl.BlockSpec((blk, 1), lambda i: (i, 0)),
            pl.BlockSpec((blk, 1), lambda i: (i, 0)),
        ],
        out_shape=[
            jax.ShapeDtypeStruct((N, F), jnp.float32),
            jax.ShapeDtypeStruct((N, 1), jnp.float32),
            jax.ShapeDtypeStruct((N, 1), jnp.float32),
        ],
    )(x, W, att_src, att_dst)


def _leaky_exp(a):
    return jnp.exp(jnp.where(a > 0, a, 0.2 * a))


def _k2a_body(asrc_hbm, adst_hbm, src_hbm, dst_hbm,
              w_hbm, dpart_hbm,
              srcc_v, dstc_v, av_v, bv_v, sbuf_v,
              shd_s, sem):
    cid = lax.axis_index("c")
    sid = lax.axis_index("s")
    base = sid * SLICE
    e0 = (cid * 16 + sid) * E_W

    zero16 = jnp.zeros((16,), jnp.float32)

    def _zero_sbuf(i, _):
        sbuf_v[pl.ds(i * 16, 16)] = zero16
        return 0
    lax.fori_loop(0, SLICE // 16, _zero_sbuf, 0)
    pltpu.sync_copy(sbuf_v, shd_s.at[pl.ds(base, SLICE)])
    plsc.subcore_barrier()

    def _wcompute(i, _):
        a16 = av_v[pl.ds(i * 16, 16)] + bv_v[pl.ds(i * 16, 16)]
        av_v[pl.ds(i * 16, 16)] = _leaky_exp(a16)
        return 0

    def _chunk(c, _):
        off = e0 + c * ECH
        pltpu.sync_copy(src_hbm.at[pl.ds(off, ECH)], srcc_v)
        pltpu.sync_copy(dst_hbm.at[pl.ds(off, ECH)], dstc_v)
        pltpu.async_copy(asrc_hbm.at[srcc_v], av_v, sem).wait()
        pltpu.async_copy(adst_hbm.at[dstc_v], bv_v, sem).wait()
        lax.fori_loop(0, ECH // 16, _wcompute, 0)
        pltpu.sync_copy(av_v, w_hbm.at[pl.ds(off, ECH)])
        pltpu.sync_copy(av_v, shd_s.at[dstc_v], add=True)
        return 0
    lax.fori_loop(0, E_W // ECH, _chunk, 0)

    plsc.subcore_barrier()
    pltpu.sync_copy(shd_s.at[pl.ds(base, SLICE)],
                    dpart_hbm.at[pl.ds(cid * NPAD + base, SLICE)])


def _sc_denoms(a_src, a_dst, src, dst):
    mesh = plsc.VectorSubcoreMesh(core_axis_name="c", subcore_axis_name="s")
    kfn = pl.kernel(
        _k2a_body,
        out_type=[
            jax.ShapeDtypeStruct((E,), jnp.float32),
            jax.ShapeDtypeStruct((2 * NPAD,), jnp.float32),
        ],
        mesh=mesh,
        compiler_params=pltpu.CompilerParams(needs_layout_passes=False),
        scratch_types=[
            pltpu.VMEM((ECH,), jnp.int32),
            pltpu.VMEM((ECH,), jnp.int32),
            pltpu.VMEM((ECH,), jnp.float32),
            pltpu.VMEM((ECH,), jnp.float32),
            pltpu.VMEM((SLICE,), jnp.float32),
            pltpu.VMEM_SHARED((NPAD,), jnp.float32),
            pltpu.SemaphoreType.DMA,
        ],
    )
    return kfn(a_src, a_dst, src, dst)


def _k2b_body(asrc_hbm, adst_hbm, src_hbm, dst_hbm, w_hbm, dpart_hbm, h2_hbm,
              alpha_e_hbm, alpha_self_hbm, dntot_hbm, s2_hbm,
              srcc_v, dstc_v, wv_v, dnv_v,
              gb0_v, gb1_v, gb2_v, sb0_v, sb1_v, sb2_v,
              asl_v, bsl_v, d0_v, d1_v,
              shS_s, sem, gsem0, gsem1, gsem2, ssem0, ssem1, ssem2):
    cid = lax.axis_index("c")
    sid = lax.axis_index("s")
    base = sid * SLICE
    e0 = sid * E_T
    crow = cid * N

    zero16 = jnp.zeros((16,), jnp.float32)
    G = ECHB // ROWBLK
    gbufs = (gb0_v, gb1_v, gb2_v)
    sbufs = (sb0_v, sb1_v, sb2_v)
    gsems = (gsem0, gsem1, gsem2)
    ssems = (ssem0, ssem1, ssem2)

    def _zero_rows(i, _):
        gb0_v[i // 4, pl.ds((i % 4) * 16, 16)] = zero16
        return 0
    lax.fori_loop(0, ROWBLK * 4, _zero_rows, 0)
    def _zero_s(k, _):
        pltpu.sync_copy(gb0_v, shS_s.at[pl.ds(base + k * ROWBLK, ROWBLK)])
        return 0
    lax.fori_loop(0, SLICE // ROWBLK, _zero_s, 0)

    pltpu.sync_copy(asrc_hbm.at[pl.ds(base, SLICE)], asl_v)
    pltpu.sync_copy(adst_hbm.at[pl.ds(base, SLICE)], bsl_v)
    pltpu.sync_copy(dpart_hbm.at[pl.ds(base, SLICE)], d0_v)
    pltpu.sync_copy(dpart_hbm.at[pl.ds(NPAD + base, SLICE)], d1_v)
    def _selfloop(i, _):
        idx = pl.ds(i * 16, 16)
        wsel = _leaky_exp(asl_v[idx] + bsl_v[idx])
        tot = d0_v[idx] + d1_v[idx] + wsel
        d0_v[idx] = tot
        asl_v[idx] = wsel / (tot + 1e-16)
        return 0
    lax.fori_loop(0, SLICE // 16, _selfloop, 0)
    pltpu.sync_copy(d0_v, dntot_hbm.at[pl.ds(base, SLICE)])
    @pl.when(cid == 0)
    def _():
        pltpu.sync_copy(asl_v, alpha_self_hbm.at[pl.ds(base, SLICE)])
    plsc.subcore_barrier()

    def _alpha(i, _):
        idx = pl.ds(i * 16, 16)
        wv_v[idx] = wv_v[idx] / (dnv_v[idx] + 1e-16)
        return 0

    def _sidx(g, half):
        return srcc_v[pl.ds(g * ROWBLK + half * 16, 16)]

    def _didx(g, half):
        return dstc_v[pl.ds(g * ROWBLK + half * 16, 16)]

    def _start_g(g, b):
        for half in range(2):
            pltpu.async_copy(h2_hbm.at[_sidx(g, half)],
                             gbufs[b].at[pl.ds(half * 16, 16)], gsems[b])

    def _wait_g(g, b):
        for half in range(2):
            pltpu.make_async_copy(h2_hbm.at[_sidx(g, half)],
                                  gbufs[b].at[pl.ds(half * 16, 16)],
                                  gsems[b]).wait()

    def _start_s(g, b):
        for half in range(2):
            pltpu.async_copy(sbufs[b].at[pl.ds(half * 16, 16)],
                             shS_s.at[_didx(g, half)], ssems[b], add=True)

    def _wait_s(g, b):
        for half in range(2):
            pltpu.make_async_copy(sbufs[b].at[pl.ds(half * 16, 16)],
                                  shS_s.at[_didx(g, half)], ssems[b]).wait()

    def _scale(g, b):
        for half in range(2):
            an = wv_v[pl.ds(g * ROWBLK + half * 16, 16)]
            for l in range(16):
                a_l = an[l]
                r = half * 16 + l
                for j in range(4):
                    sbufs[b][r, pl.ds(j * 16, 16)] = (
                        gbufs[b][r, pl.ds(j * 16, 16)] * a_l)
        return 0

    def _srcoff(i, _):
        idx = pl.ds(i * 16, 16)
        srcc_v[idx] = srcc_v[idx] + crow
        return 0

    def _chunk(c, _):
        off = e0 + c * ECHB
        pltpu.sync_copy(src_hbm.at[pl.ds(off, ECHB)], srcc_v)
        pltpu.sync_copy(dst_hbm.at[pl.ds(off, ECHB)], dstc_v)
        pltpu.sync_copy(w_hbm.at[pl.ds(off, ECHB)], wv_v)
        pltpu.async_copy(dntot_hbm.at[dstc_v], dnv_v, sem).wait()
        lax.fori_loop(0, ECHB // 16, _alpha, 0)
        lax.fori_loop(0, ECHB // 16, _srcoff, 0)
        @pl.when(cid == 0)
        def _():
            pltpu.sync_copy(wv_v, alpha_e_hbm.at[pl.ds(off, ECHB)])

        for b in range(3):
            _start_g(b, b)
        for b in range(3):
            _wait_g(b, b)
            _scale(b, b)
            _start_g(b + 3, b)
            _start_s(b, b)

        def _pipe(k, _):
            for b in range(3):
                g = 3 * k + b
                _wait_g(g, b)
                _wait_s(g - 3, b)
                _scale(g, b)
                @pl.when(g + 3 < G)
                def _():
                    _start_g(g + 3, b)
                _start_s(g, b)
            return 0
        lax.fori_loop(1, G // 3, _pipe, 0)

        for b, g in ((0, G - 2), (1, G - 1)):
            _wait_g(g, b)
            _wait_s(g - 3, b)
            _scale(g, b)
            _start_s(g, b)
        _wait_s(G - 3, 2)
        _wait_s(G - 2, 0)
        _wait_s(G - 1, 1)
        return 0

    lax.fori_loop(0, E_T // ECHB, _chunk, 0)

    plsc.subcore_barrier()
    pltpu.sync_copy(shS_s.at[pl.ds(base, SLICE)],
                    s2_hbm.at[pl.ds(cid * NPAD + base, SLICE)])


def _sc_rows(a_src, a_dst, src, dst, w_e, dpart, h2):
    mesh = plsc.VectorSubcoreMesh(core_axis_name="c", subcore_axis_name="s")
    kfn = pl.kernel(
        _k2b_body,
        out_type=[
            jax.ShapeDtypeStruct((E,), jnp.float32),
            jax.ShapeDtypeStruct((NPAD,), jnp.float32),
            jax.ShapeDtypeStruct((NPAD,), jnp.float32),
            jax.ShapeDtypeStruct((2 * NPAD, FH), jnp.float32),
        ],
        mesh=mesh,
        compiler_params=pltpu.CompilerParams(needs_layout_passes=False,
                                             use_tc_tiling_on_sc=False),
        scratch_types=[
            pltpu.VMEM((ECHB,), jnp.int32),
            pltpu.VMEM((ECHB,), jnp.int32),
            pltpu.VMEM((ECHB,), jnp.float32),
            pltpu.VMEM((ECHB,), jnp.float32),
            pltpu.VMEM((ROWBLK, FH), jnp.float32),
            pltpu.VMEM((ROWBLK, FH), jnp.float32),
            pltpu.VMEM((ROWBLK, FH), jnp.float32),
            pltpu.VMEM((ROWBLK, FH), jnp.float32),
            pltpu.VMEM((ROWBLK, FH), jnp.float32),
            pltpu.VMEM((ROWBLK, FH), jnp.float32),
            pltpu.VMEM((SLICE,), jnp.float32),
            pltpu.VMEM((SLICE,), jnp.float32),
            pltpu.VMEM((SLICE,), jnp.float32),
            pltpu.VMEM((SLICE,), jnp.float32),
            pltpu.VMEM_SHARED((NPAD, FH), jnp.float32),
            pltpu.SemaphoreType.DMA,
            pltpu.SemaphoreType.DMA,
            pltpu.SemaphoreType.DMA,
            pltpu.SemaphoreType.DMA,
            pltpu.SemaphoreType.DMA,
            pltpu.SemaphoreType.DMA,
            pltpu.SemaphoreType.DMA,
        ],
    )
    return kfn(a_src, a_dst, src, dst, w_e, dpart, h2)


def _post_body(s_ref, asel_ref, h_ref, b_ref, o_ref):
    s = jnp.concatenate([s_ref[0], s_ref[1]], axis=-1)
    v = s + asel_ref[...] * h_ref[...] + b_ref[...]
    o_ref[...] = jnp.where(v > 0, v, jnp.exp(jnp.minimum(v, 0.0)) - 1.0)


def _tc_epilogue(s2, alpha_self, h, bias):
    blk = N // 10
    return pl.pallas_call(
        _post_body,
        grid=(10,),
        in_specs=[
            pl.BlockSpec((2, blk, FH), lambda i: (0, i, 0)),
            pl.BlockSpec((blk, 1), lambda i: (i, 0)),
            pl.BlockSpec((blk, F), lambda i: (i, 0)),
            pl.BlockSpec((1, F), lambda i: (0, 0)),
        ],
        out_specs=pl.BlockSpec((blk, F), lambda i: (i, 0)),
        out_shape=jax.ShapeDtypeStruct((N, F), jnp.float32),
    )(s2, alpha_self, h, bias)


@jax.jit
def kernel(x, edge_index, batch, W, att_src, att_dst, bias):
    src = edge_index[0]
    dst = edge_index[1]
    h, a_src2, a_dst2 = _tc_prologue(x, W, att_src.reshape(1, F), att_dst.reshape(1, F))
    a_src = jnp.pad(a_src2.reshape(N), (0, NPAD - N))
    a_dst = jnp.pad(a_dst2.reshape(N), (0, NPAD - N))

    w_e, dpart = _sc_denoms(a_src, a_dst, src, dst)

    h2 = jnp.transpose(h.reshape(N, 2, FH), (1, 0, 2)).reshape(2 * N, FH)
    alpha_e, alpha_self_pad, _dn, s2 = _sc_rows(
        a_src, a_dst, src, dst, w_e, dpart, h2)

    s_halves = s2.reshape(2, NPAD, FH)[:, :N, :]
    out = _tc_epilogue(s_halves, alpha_self_pad[:N].reshape(N, 1), h,
                       bias.reshape(1, F))

    loop = jnp.arange(N, dtype=edge_index.dtype)
    ei = jnp.concatenate([edge_index, jnp.stack([loop, loop], axis=0)], axis=1)
    alpha = jnp.concatenate([alpha_e, alpha_self_pad[:N]]).reshape(E + N, 1)
    return out, ei, alpha


# speedup vs baseline: 32.1812x; 1.1435x over previous
import jax
import jax.numpy as jnp
from jax import lax
from jax.experimental import pallas as pl
from jax.experimental.pallas import tpu as pltpu
from jax.experimental.pallas import tpu_sc as plsc

N = 10000
E = 320000
F = 128
FH = F // 2
NPAD = 10240
SLICE = NPAD // 16
E_T = E // 16
E_W = E // 32
ECH = 2000
ECHB = 4000
ROWBLK = 32


def _pre_body(x_ref, w_ref, as_ref, ad_ref, h_ref, asrc_ref, adst_ref):
    h = jnp.dot(x_ref[...], w_ref[...], preferred_element_type=jnp.float32)
    h_ref[...] = h
    asrc_ref[...] = jnp.sum(h * as_ref[...], axis=1, keepdims=True)
    adst_ref[...] = jnp.sum(h * ad_ref[...], axis=1, keepdims=True)


def _tc_prologue(x, W, att_src, att_dst):
    blk = N // 10
    return pl.pallas_call(
        _pre_body,
        grid=(10,),
        in_specs=[
            pl.BlockSpec((blk, F), lambda i: (i, 0)),
            pl.BlockSpec((F, F), lambda i: (0, 0)),
            pl.BlockSpec((1, F), lambda i: (0, 0)),
            pl.BlockSpec((1, F), lambda i: (0, 0)),
        ],
        out_specs=[
            pl.BlockSpec((blk, F), lambda i: (i, 0)),
            pl.BlockSpec((blk, 1), lambda i: (i, 0)),
            pl.BlockSpec((blk, 1), lambda i: (i, 0)),
        ],
        out_shape=[
            jax.ShapeDtypeStruct((N, F), jnp.float32),
            jax.ShapeDtypeStruct((N, 1), jnp.float32),
            jax.ShapeDtypeStruct((N, 1), jnp.float32),
        ],
    )(x, W, att_src, att_dst)


def _leaky_exp(a):
    return jnp.exp(jnp.where(a > 0, a, 0.2 * a))


def _k2a_body(asrc_hbm, adst_hbm, src_hbm, dst_hbm,
              w_hbm, dpart_hbm,
              srcc0_v, srcc1_v, dstc0_v, dstc1_v,
              av0_v, av1_v, bv0_v, bv1_v, sbuf_v,
              shd_s, sema0, sema1, semb0, semb1):
    cid = lax.axis_index("c")
    sid = lax.axis_index("s")
    base = sid * SLICE
    e0 = (cid * 16 + sid) * E_W

    zero16 = jnp.zeros((16,), jnp.float32)
    srccs = (srcc0_v, srcc1_v)
    dstcs = (dstc0_v, dstc1_v)
    avs = (av0_v, av1_v)
    bvs = (bv0_v, bv1_v)
    semas = (sema0, sema1)
    sembs = (semb0, semb1)
    NCH = E_W // ECH

    def _zero_sbuf(i, _):
        sbuf_v[pl.ds(i * 16, 16)] = zero16
        return 0
    lax.fori_loop(0, SLICE // 16, _zero_sbuf, 0)
    pltpu.sync_copy(sbuf_v, shd_s.at[pl.ds(base, SLICE)])
    plsc.subcore_barrier()

    def _make_wcompute(p):
        def _wcompute(i, _):
            a16 = avs[p][pl.ds(i * 16, 16)] + bvs[p][pl.ds(i * 16, 16)]
            avs[p][pl.ds(i * 16, 16)] = _leaky_exp(a16)
            return 0
        return _wcompute

    def _prefetch(c, p):
        off = e0 + c * ECH
        pltpu.sync_copy(src_hbm.at[pl.ds(off, ECH)], srccs[p])
        pltpu.sync_copy(dst_hbm.at[pl.ds(off, ECH)], dstcs[p])
        pltpu.async_copy(asrc_hbm.at[srccs[p]], avs[p], semas[p])
        pltpu.async_copy(adst_hbm.at[dstcs[p]], bvs[p], sembs[p])

    _prefetch(0, 0)
    for c in range(NCH):
        p = c % 2
        if c + 1 < NCH:
            _prefetch(c + 1, (c + 1) % 2)
        off = e0 + c * ECH
        pltpu.make_async_copy(asrc_hbm.at[srccs[p]], avs[p], semas[p]).wait()
        pltpu.make_async_copy(adst_hbm.at[dstcs[p]], bvs[p], sembs[p]).wait()
        lax.fori_loop(0, ECH // 16, _make_wcompute(p), 0)
        pltpu.sync_copy(avs[p], w_hbm.at[pl.ds(off, ECH)])
        pltpu.sync_copy(avs[p], shd_s.at[dstcs[p]], add=True)

    plsc.subcore_barrier()
    pltpu.sync_copy(shd_s.at[pl.ds(base, SLICE)],
                    dpart_hbm.at[pl.ds(cid * NPAD + base, SLICE)])


def _sc_denoms(a_src, a_dst, src, dst):
    mesh = plsc.VectorSubcoreMesh(core_axis_name="c", subcore_axis_name="s")
    kfn = pl.kernel(
        _k2a_body,
        out_type=[
            jax.ShapeDtypeStruct((E,), jnp.float32),
            jax.ShapeDtypeStruct((2 * NPAD,), jnp.float32),
        ],
        mesh=mesh,
        compiler_params=pltpu.CompilerParams(needs_layout_passes=False),
        scratch_types=[
            pltpu.VMEM((ECH,), jnp.int32),
            pltpu.VMEM((ECH,), jnp.int32),
            pltpu.VMEM((ECH,), jnp.int32),
            pltpu.VMEM((ECH,), jnp.int32),
            pltpu.VMEM((ECH,), jnp.float32),
            pltpu.VMEM((ECH,), jnp.float32),
            pltpu.VMEM((ECH,), jnp.float32),
            pltpu.VMEM((ECH,), jnp.float32),
            pltpu.VMEM((SLICE,), jnp.float32),
            pltpu.VMEM_SHARED((NPAD,), jnp.float32),
            pltpu.SemaphoreType.DMA,
            pltpu.SemaphoreType.DMA,
            pltpu.SemaphoreType.DMA,
            pltpu.SemaphoreType.DMA,
        ],
    )
    return kfn(a_src, a_dst, src, dst)


def _k2b_body(asrc_hbm, adst_hbm, src_hbm, dst_hbm, w_hbm, dpart_hbm, h2_hbm,
              alpha_e_hbm, dntot_hbm, s2_hbm,
              srcc_v, dstc_v, wv_v, dnv_v,
              gb0_v, gb1_v, gb2_v, gb3_v, gb4_v, gb5_v,
              sb0_v, sb1_v, sb2_v,
              asl_v, bsl_v, d0_v, d1_v,
              shS_s, sem, gsem0, gsem1, gsem2, gsem3, gsem4, gsem5,
              ssem0, ssem1, ssem2):
    cid = lax.axis_index("c")
    sid = lax.axis_index("s")
    base = sid * SLICE
    e0 = sid * E_T
    crow = cid * N

    zero16 = jnp.zeros((16,), jnp.float32)
    G = ECHB // ROWBLK
    gbufs = (gb0_v, gb1_v, gb2_v, gb3_v, gb4_v, gb5_v)
    sbufs = (sb0_v, sb1_v, sb2_v)
    gsems = (gsem0, gsem1, gsem2, gsem3, gsem4, gsem5)
    ssems = (ssem0, ssem1, ssem2)

    def _zero_rows(i, _):
        gb0_v[i // 4, pl.ds((i % 4) * 16, 16)] = zero16
        return 0
    lax.fori_loop(0, ROWBLK * 4, _zero_rows, 0)
    def _zero_s(k, _):
        pltpu.sync_copy(gb0_v, shS_s.at[pl.ds(base + k * ROWBLK, ROWBLK)])
        return 0
    lax.fori_loop(0, SLICE // ROWBLK, _zero_s, 0)

    pltpu.sync_copy(asrc_hbm.at[pl.ds(base, SLICE)], asl_v)
    pltpu.sync_copy(adst_hbm.at[pl.ds(base, SLICE)], bsl_v)
    pltpu.sync_copy(dpart_hbm.at[pl.ds(base, SLICE)], d0_v)
    pltpu.sync_copy(dpart_hbm.at[pl.ds(NPAD + base, SLICE)], d1_v)
    def _selfloop(i, _):
        idx = pl.ds(i * 16, 16)
        wsel = _leaky_exp(asl_v[idx] + bsl_v[idx])
        tot = d0_v[idx] + d1_v[idx] + wsel
        d0_v[idx] = tot
        asl_v[idx] = wsel / (tot + 1e-16)
        return 0
    lax.fori_loop(0, SLICE // 16, _selfloop, 0)
    pltpu.sync_copy(d0_v, dntot_hbm.at[pl.ds(base, SLICE)])
    @pl.when(cid == 0)
    def _():
        pltpu.sync_copy(asl_v, alpha_e_hbm.at[pl.ds(E + base, SLICE)])
    plsc.subcore_barrier()

    def _alpha(i, _):
        idx = pl.ds(i * 16, 16)
        wv_v[idx] = wv_v[idx] / (dnv_v[idx] + 1e-16)
        return 0

    def _sidx(g, half):
        return srcc_v[pl.ds(g * ROWBLK + half * 16, 16)]

    def _didx(g, half):
        return dstc_v[pl.ds(g * ROWBLK + half * 16, 16)]

    def _start_g(g, b):
        for half in range(2):
            pltpu.async_copy(h2_hbm.at[_sidx(g, half)],
                             gbufs[b].at[pl.ds(half * 16, 16)], gsems[b])

    def _wait_g(g, b):
        for half in range(2):
            pltpu.make_async_copy(h2_hbm.at[_sidx(g, half)],
                                  gbufs[b].at[pl.ds(half * 16, 16)],
                                  gsems[b]).wait()

    def _start_s(g, sb):
        for half in range(2):
            pltpu.async_copy(sbufs[sb].at[pl.ds(half * 16, 16)],
                             shS_s.at[_didx(g, half)], ssems[sb], add=True)

    def _wait_s(g, sb):
        for half in range(2):
            pltpu.make_async_copy(sbufs[sb].at[pl.ds(half * 16, 16)],
                                  shS_s.at[_didx(g, half)], ssems[sb]).wait()

    def _scale(g, b, sb):
        for half in range(2):
            an = wv_v[pl.ds(g * ROWBLK + half * 16, 16)]
            for l in range(16):
                a_l = an[l]
                r = half * 16 + l
                for j in range(4):
                    sbufs[sb][r, pl.ds(j * 16, 16)] = (
                        gbufs[b][r, pl.ds(j * 16, 16)] * a_l)
        return 0

    def _srcoff(i, _):
        idx = pl.ds(i * 16, 16)
        srcc_v[idx] = srcc_v[idx] + crow
        return 0

    def _chunk(c, _):
        off = e0 + c * ECHB
        pltpu.sync_copy(src_hbm.at[pl.ds(off, ECHB)], srcc_v)
        pltpu.sync_copy(dst_hbm.at[pl.ds(off, ECHB)], dstc_v)
        pltpu.sync_copy(w_hbm.at[pl.ds(off, ECHB)], wv_v)
        pltpu.async_copy(dntot_hbm.at[dstc_v], dnv_v, sem).wait()
        lax.fori_loop(0, ECHB // 16, _alpha, 0)
        lax.fori_loop(0, ECHB // 16, _srcoff, 0)
        @pl.when(cid == 0)
        def _():
            pltpu.sync_copy(wv_v, alpha_e_hbm.at[pl.ds(off, ECHB)])

        for b in range(6):
            _start_g(b, b)
        for b in range(6):
            _wait_g(b, b)
            if b >= 3:
                _wait_s(b - 3, b % 3)
            _scale(b, b, b % 3)
            _start_g(b + 6, b)
            _start_s(b, b % 3)

        def _pipe(k, _):
            for b in range(6):
                g = 6 * k + b
                _wait_g(g, b)
                _wait_s(g - 3, b % 3)
                _scale(g, b, b % 3)
                @pl.when(g + 6 < G)
                def _():
                    _start_g(g + 6, b)
                _start_s(g, b % 3)
            return 0
        lax.fori_loop(1, G // 6, _pipe, 0)

        for g in range(6 * (G // 6), G):
            b = g % 6
            _wait_g(g, b)
            _wait_s(g - 3, b % 3)
            _scale(g, b, b % 3)
            _start_s(g, b % 3)
        for g in range(G - 3, G):
            _wait_s(g, g % 3)
        return 0

    lax.fori_loop(0, E_T // ECHB, _chunk, 0)

    plsc.subcore_barrier()
    pltpu.sync_copy(shS_s.at[pl.ds(base, SLICE)],
                    s2_hbm.at[pl.ds(cid * NPAD + base, SLICE)])


def _sc_rows(a_src, a_dst, src, dst, w_e, dpart, h2):
    mesh = plsc.VectorSubcoreMesh(core_axis_name="c", subcore_axis_name="s")
    kfn = pl.kernel(
        _k2b_body,
        out_type=[
            jax.ShapeDtypeStruct((E + NPAD,), jnp.float32),
            jax.ShapeDtypeStruct((NPAD,), jnp.float32),
            jax.ShapeDtypeStruct((2 * NPAD, FH), jnp.float32),
        ],
        mesh=mesh,
        compiler_params=pltpu.CompilerParams(needs_layout_passes=False,
                                             use_tc_tiling_on_sc=False),
        scratch_types=[
            pltpu.VMEM((ECHB,), jnp.int32),
            pltpu.VMEM((ECHB,), jnp.int32),
            pltpu.VMEM((ECHB,), jnp.float32),
            pltpu.VMEM((ECHB,), jnp.float32),
            pltpu.VMEM((ROWBLK, FH), jnp.float32),
            pltpu.VMEM((ROWBLK, FH), jnp.float32),
            pltpu.VMEM((ROWBLK, FH), jnp.float32),
            pltpu.VMEM((ROWBLK, FH), jnp.float32),
            pltpu.VMEM((ROWBLK, FH), jnp.float32),
            pltpu.VMEM((ROWBLK, FH), jnp.float32),
            pltpu.VMEM((ROWBLK, FH), jnp.float32),
            pltpu.VMEM((ROWBLK, FH), jnp.float32),
            pltpu.VMEM((ROWBLK, FH), jnp.float32),
            pltpu.VMEM((SLICE,), jnp.float32),
            pltpu.VMEM((SLICE,), jnp.float32),
            pltpu.VMEM((SLICE,), jnp.float32),
            pltpu.VMEM((SLICE,), jnp.float32),
            pltpu.VMEM_SHARED((NPAD, FH), jnp.float32),
            pltpu.SemaphoreType.DMA,
            pltpu.SemaphoreType.DMA,
            pltpu.SemaphoreType.DMA,
            pltpu.SemaphoreType.DMA,
            pltpu.SemaphoreType.DMA,
            pltpu.SemaphoreType.DMA,
            pltpu.SemaphoreType.DMA,
            pltpu.SemaphoreType.DMA,
            pltpu.SemaphoreType.DMA,
            pltpu.SemaphoreType.DMA,
        ],
    )
    return kfn(a_src, a_dst, src, dst, w_e, dpart, h2)


def _post_body(s_ref, asel_ref, h_ref, b_ref, o_ref):
    s = jnp.concatenate([s_ref[0], s_ref[1]], axis=-1)
    v = s + asel_ref[...] * h_ref[...] + b_ref[...]
    o_ref[...] = jnp.where(v > 0, v, jnp.exp(jnp.minimum(v, 0.0)) - 1.0)


def _tc_epilogue(s2, alpha_self, h, bias):
    blk = N // 10
    return pl.pallas_call(
        _post_body,
        grid=(10,),
        in_specs=[
            pl.BlockSpec((2, blk, FH), lambda i: (0, i, 0)),
            pl.BlockSpec((blk, 1), lambda i: (i, 0)),
            pl.BlockSpec((blk, F), lambda i: (i, 0)),
            pl.BlockSpec((1, F), lambda i: (0, 0)),
        ],
        out_specs=pl.BlockSpec((blk, F), lambda i: (i, 0)),
        out_shape=jax.ShapeDtypeStruct((N, F), jnp.float32),
    )(s2, alpha_self, h, bias)


@jax.jit
def kernel(x, edge_index, batch, W, att_src, att_dst, bias):
    src = edge_index[0]
    dst = edge_index[1]
    h, a_src2, a_dst2 = _tc_prologue(x, W, att_src.reshape(1, F), att_dst.reshape(1, F))
    a_src = jnp.pad(a_src2.reshape(N), (0, NPAD - N))
    a_dst = jnp.pad(a_dst2.reshape(N), (0, NPAD - N))

    w_e, dpart = _sc_denoms(a_src, a_dst, src, dst)

    h2 = jnp.transpose(h.reshape(N, 2, FH), (1, 0, 2)).reshape(2 * N, FH)
    alpha_all, _dn, s2 = _sc_rows(
        a_src, a_dst, src, dst, w_e, dpart, h2)

    s_halves = s2.reshape(2, NPAD, FH)[:, :N, :]
    out = _tc_epilogue(s_halves, alpha_all[E:E + N].reshape(N, 1), h,
                       bias.reshape(1, F))

    loop = jnp.arange(N, dtype=edge_index.dtype)
    ei = jnp.concatenate([edge_index, jnp.stack([loop, loop], axis=0)], axis=1)
    alpha = alpha_all[:E + N].reshape(E + N, 1)
    return out, ei, alpha


# speedup vs baseline: 34.6817x; 1.0777x over previous
import jax
import jax.numpy as jnp
from jax import lax
from jax.experimental import pallas as pl
from jax.experimental.pallas import tpu as pltpu
from jax.experimental.pallas import tpu_sc as plsc

N = 10000
E = 320000
F = 128
FH = F // 2
NPAD = 10240
SLICE = NPAD // 16
E_T = E // 16
E_W = E // 32
ECH = 2000
ECHB = 4000
ROWBLK = 32


def _pre_body(x_ref, w_ref, as_ref, ad_ref, h2_ref, asrc_ref, adst_ref):
    half = pl.program_id(1)
    hh = jnp.dot(x_ref[...], w_ref[0], preferred_element_type=jnp.float32)
    h2_ref[...] = hh
    pa = jnp.sum(hh * as_ref[0], axis=1, keepdims=True)
    pb = jnp.sum(hh * ad_ref[0], axis=1, keepdims=True)

    @pl.when(half == 0)
    def _():
        asrc_ref[...] = pa
        adst_ref[...] = pb

    @pl.when(half != 0)
    def _():
        asrc_ref[...] = asrc_ref[...] + pa
        adst_ref[...] = adst_ref[...] + pb


def _tc_prologue(x, W, att_src, att_dst):
    blk = N // 10
    return pl.pallas_call(
        _pre_body,
        grid=(10, 2),
        in_specs=[
            pl.BlockSpec((blk, F), lambda i, j: (i, 0)),
            pl.BlockSpec((1, F, FH), lambda i, j: (j, 0, 0)),
            pl.BlockSpec((1, 1, FH), lambda i, j: (j, 0, 0)),
            pl.BlockSpec((1, 1, FH), lambda i, j: (j, 0, 0)),
        ],
        out_specs=[
            pl.BlockSpec((blk, FH), lambda i, j: (j * 10 + i, 0)),
            pl.BlockSpec((blk, 1), lambda i, j: (i, 0)),
            pl.BlockSpec((blk, 1), lambda i, j: (i, 0)),
        ],
        out_shape=[
            jax.ShapeDtypeStruct((2 * N, FH), jnp.float32),
            jax.ShapeDtypeStruct((N, 1), jnp.float32),
            jax.ShapeDtypeStruct((N, 1), jnp.float32),
        ],
    )(x, jnp.stack([W[:, :FH], W[:, FH:]]),
      jnp.stack([att_src[:, :FH], att_src[:, FH:]]),
      jnp.stack([att_dst[:, :FH], att_dst[:, FH:]]))


def _leaky_exp(a):
    return jnp.exp(jnp.where(a > 0, a, 0.2 * a))


def _k2a_body(asrc_hbm, adst_hbm, src_hbm, dst_hbm,
              w_hbm, dpart_hbm,
              srcc0_v, srcc1_v, dstc0_v, dstc1_v,
              av0_v, av1_v, bv0_v, bv1_v, sbuf_v,
              shd_s, sema0, sema1, semb0, semb1):
    cid = lax.axis_index("c")
    sid = lax.axis_index("s")
    base = sid * SLICE
    e0 = (cid * 16 + sid) * E_W

    zero16 = jnp.zeros((16,), jnp.float32)
    srccs = (srcc0_v, srcc1_v)
    dstcs = (dstc0_v, dstc1_v)
    avs = (av0_v, av1_v)
    bvs = (bv0_v, bv1_v)
    semas = (sema0, sema1)
    sembs = (semb0, semb1)
    NCH = E_W // ECH

    def _zero_sbuf(i, _):
        sbuf_v[pl.ds(i * 16, 16)] = zero16
        return 0
    lax.fori_loop(0, SLICE // 16, _zero_sbuf, 0)
    pltpu.sync_copy(sbuf_v, shd_s.at[pl.ds(base, SLICE)])
    plsc.subcore_barrier()

    def _make_wcompute(p):
        def _wcompute(i, _):
            a16 = avs[p][pl.ds(i * 16, 16)] + bvs[p][pl.ds(i * 16, 16)]
            avs[p][pl.ds(i * 16, 16)] = _leaky_exp(a16)
            return 0
        return _wcompute

    def _prefetch(c, p):
        off = e0 + c * ECH
        pltpu.sync_copy(src_hbm.at[pl.ds(off, ECH)], srccs[p])
        pltpu.sync_copy(dst_hbm.at[pl.ds(off, ECH)], dstcs[p])
        pltpu.async_copy(asrc_hbm.at[srccs[p]], avs[p], semas[p])
        pltpu.async_copy(adst_hbm.at[dstcs[p]], bvs[p], sembs[p])

    _prefetch(0, 0)
    for c in range(NCH):
        p = c % 2
        if c + 1 < NCH:
            _prefetch(c + 1, (c + 1) % 2)
        off = e0 + c * ECH
        pltpu.make_async_copy(asrc_hbm.at[srccs[p]], avs[p], semas[p]).wait()
        pltpu.make_async_copy(adst_hbm.at[dstcs[p]], bvs[p], sembs[p]).wait()
        lax.fori_loop(0, ECH // 16, _make_wcompute(p), 0)
        pltpu.sync_copy(avs[p], w_hbm.at[pl.ds(off, ECH)])
        pltpu.sync_copy(avs[p], shd_s.at[dstcs[p]], add=True)

    plsc.subcore_barrier()
    pltpu.sync_copy(shd_s.at[pl.ds(base, SLICE)],
                    dpart_hbm.at[pl.ds(cid * NPAD + base, SLICE)])


def _sc_denoms(a_src, a_dst, src, dst):
    mesh = plsc.VectorSubcoreMesh(core_axis_name="c", subcore_axis_name="s")
    kfn = pl.kernel(
        _k2a_body,
        out_type=[
            jax.ShapeDtypeStruct((E,), jnp.float32),
            jax.ShapeDtypeStruct((2 * NPAD,), jnp.float32),
        ],
        mesh=mesh,
        compiler_params=pltpu.CompilerParams(needs_layout_passes=False),
        scratch_types=[
            pltpu.VMEM((ECH,), jnp.int32),
            pltpu.VMEM((ECH,), jnp.int32),
            pltpu.VMEM((ECH,), jnp.int32),
            pltpu.VMEM((ECH,), jnp.int32),
            pltpu.VMEM((ECH,), jnp.float32),
            pltpu.VMEM((ECH,), jnp.float32),
            pltpu.VMEM((ECH,), jnp.float32),
            pltpu.VMEM((ECH,), jnp.float32),
            pltpu.VMEM((SLICE,), jnp.float32),
            pltpu.VMEM_SHARED((NPAD,), jnp.float32),
            pltpu.SemaphoreType.DMA,
            pltpu.SemaphoreType.DMA,
            pltpu.SemaphoreType.DMA,
            pltpu.SemaphoreType.DMA,
        ],
    )
    return kfn(a_src, a_dst, src, dst)


def _k2b_body(asrc_hbm, adst_hbm, src_hbm, dst_hbm, w_hbm, dpart_hbm, h2_hbm,
              alpha_e_hbm, dntot_hbm, s2_hbm,
              srcc_v, dstc_v, wv_v, dnv_v,
              gb0_v, gb1_v, gb2_v, gb3_v, gb4_v, gb5_v,
              sb0_v, sb1_v, sb2_v,
              asl_v, bsl_v, d0_v, d1_v,
              shS_s, sem, gsem0, gsem1, gsem2, gsem3, gsem4, gsem5,
              ssem0, ssem1, ssem2):
    cid = lax.axis_index("c")
    sid = lax.axis_index("s")
    base = sid * SLICE
    e0 = sid * E_T
    crow = cid * N

    zero16 = jnp.zeros((16,), jnp.float32)
    G = ECHB // ROWBLK
    gbufs = (gb0_v, gb1_v, gb2_v, gb3_v, gb4_v, gb5_v)
    sbufs = (sb0_v, sb1_v, sb2_v)
    gsems = (gsem0, gsem1, gsem2, gsem3, gsem4, gsem5)
    ssems = (ssem0, ssem1, ssem2)

    def _zero_rows(i, _):
        gb0_v[i // 4, pl.ds((i % 4) * 16, 16)] = zero16
        return 0
    lax.fori_loop(0, ROWBLK * 4, _zero_rows, 0)
    def _zero_s(k, _):
        pltpu.sync_copy(gb0_v, shS_s.at[pl.ds(base + k * ROWBLK, ROWBLK)])
        return 0
    lax.fori_loop(0, SLICE // ROWBLK, _zero_s, 0)

    pltpu.sync_copy(asrc_hbm.at[pl.ds(base, SLICE)], asl_v)
    pltpu.sync_copy(adst_hbm.at[pl.ds(base, SLICE)], bsl_v)
    pltpu.sync_copy(dpart_hbm.at[pl.ds(base, SLICE)], d0_v)
    pltpu.sync_copy(dpart_hbm.at[pl.ds(NPAD + base, SLICE)], d1_v)
    def _selfloop(i, _):
        idx = pl.ds(i * 16, 16)
        wsel = _leaky_exp(asl_v[idx] + bsl_v[idx])
        tot = d0_v[idx] + d1_v[idx] + wsel
        d0_v[idx] = tot
        asl_v[idx] = wsel / (tot + 1e-16)
        return 0
    lax.fori_loop(0, SLICE // 16, _selfloop, 0)
    pltpu.sync_copy(d0_v, dntot_hbm.at[pl.ds(base, SLICE)])
    @pl.when(cid == 0)
    def _():
        pltpu.sync_copy(asl_v, alpha_e_hbm.at[pl.ds(E + base, SLICE)])
    plsc.subcore_barrier()

    def _alpha(i, _):
        idx = pl.ds(i * 16, 16)
        wv_v[idx] = wv_v[idx] / (dnv_v[idx] + 1e-16)
        srcc_v[idx] = srcc_v[idx] + crow
        return 0

    def _sidx(g, half):
        return srcc_v[pl.ds(g * ROWBLK + half * 16, 16)]

    def _didx(g, half):
        return dstc_v[pl.ds(g * ROWBLK + half * 16, 16)]

    def _start_g(g, b):
        for half in range(2):
            pltpu.async_copy(h2_hbm.at[_sidx(g, half)],
                             gbufs[b].at[pl.ds(half * 16, 16)], gsems[b])

    def _wait_g(g, b):
        for half in range(2):
            pltpu.make_async_copy(h2_hbm.at[_sidx(g, half)],
                                  gbufs[b].at[pl.ds(half * 16, 16)],
                                  gsems[b]).wait()

    def _start_s(g, sb):
        for half in range(2):
            pltpu.async_copy(sbufs[sb].at[pl.ds(half * 16, 16)],
                             shS_s.at[_didx(g, half)], ssems[sb], add=True)

    def _wait_s(g, sb):
        for half in range(2):
            pltpu.make_async_copy(sbufs[sb].at[pl.ds(half * 16, 16)],
                                  shS_s.at[_didx(g, half)], ssems[sb]).wait()

    def _scale(g, b, sb):
        for half in range(2):
            an = wv_v[pl.ds(g * ROWBLK + half * 16, 16)]
            for l in range(16):
                a_l = an[l]
                r = half * 16 + l
                for j in range(4):
                    sbufs[sb][r, pl.ds(j * 16, 16)] = (
                        gbufs[b][r, pl.ds(j * 16, 16)] * a_l)
        return 0

    def _chunk(c, _):
        off = e0 + c * ECHB
        pltpu.sync_copy(src_hbm.at[pl.ds(off, ECHB)], srcc_v)
        pltpu.sync_copy(dst_hbm.at[pl.ds(off, ECHB)], dstc_v)
        pltpu.sync_copy(w_hbm.at[pl.ds(off, ECHB)], wv_v)
        pltpu.async_copy(dntot_hbm.at[dstc_v], dnv_v, sem).wait()
        lax.fori_loop(0, ECHB // 16, _alpha, 0)
        @pl.when(cid == 0)
        def _():
            pltpu.sync_copy(wv_v, alpha_e_hbm.at[pl.ds(off, ECHB)])

        for b in range(6):
            _start_g(b, b)
        for b in range(6):
            _wait_g(b, b)
            if b >= 3:
                _wait_s(b - 3, b % 3)
            _scale(b, b, b % 3)
            _start_g(b + 6, b)
            _start_s(b, b % 3)

        def _pipe(k, _):
            for b in range(6):
                g = 6 * k + b
                _wait_g(g, b)
                _wait_s(g - 3, b % 3)
                _scale(g, b, b % 3)
                @pl.when(g + 6 < G)
                def _():
                    _start_g(g + 6, b)
                _start_s(g, b % 3)
            return 0
        lax.fori_loop(1, G // 6, _pipe, 0)

        for g in range(6 * (G // 6), G):
            b = g % 6
            _wait_g(g, b)
            _wait_s(g - 3, b % 3)
            _scale(g, b, b % 3)
            _start_s(g, b % 3)
        for g in range(G - 3, G):
            _wait_s(g, g % 3)
        return 0

    lax.fori_loop(0, E_T // ECHB, _chunk, 0)

    plsc.subcore_barrier()
    pltpu.sync_copy(shS_s.at[pl.ds(base, SLICE)],
                    s2_hbm.at[pl.ds(cid * NPAD + base, SLICE)])


def _sc_rows(a_src, a_dst, src, dst, w_e, dpart, h2):
    mesh = plsc.VectorSubcoreMesh(core_axis_name="c", subcore_axis_name="s")
    kfn = pl.kernel(
        _k2b_body,
        out_type=[
            jax.ShapeDtypeStruct((E + NPAD,), jnp.float32),
            jax.ShapeDtypeStruct((NPAD,), jnp.float32),
            jax.ShapeDtypeStruct((2 * NPAD, FH), jnp.float32),
        ],
        mesh=mesh,
        compiler_params=pltpu.CompilerParams(needs_layout_passes=False,
                                             use_tc_tiling_on_sc=False),
        scratch_types=[
            pltpu.VMEM((ECHB,), jnp.int32),
            pltpu.VMEM((ECHB,), jnp.int32),
            pltpu.VMEM((ECHB,), jnp.float32),
            pltpu.VMEM((ECHB,), jnp.float32),
            pltpu.VMEM((ROWBLK, FH), jnp.float32),
            pltpu.VMEM((ROWBLK, FH), jnp.float32),
            pltpu.VMEM((ROWBLK, FH), jnp.float32),
            pltpu.VMEM((ROWBLK, FH), jnp.float32),
            pltpu.VMEM((ROWBLK, FH), jnp.float32),
            pltpu.VMEM((ROWBLK, FH), jnp.float32),
            pltpu.VMEM((ROWBLK, FH), jnp.float32),
            pltpu.VMEM((ROWBLK, FH), jnp.float32),
            pltpu.VMEM((ROWBLK, FH), jnp.float32),
            pltpu.VMEM((SLICE,), jnp.float32),
            pltpu.VMEM((SLICE,), jnp.float32),
            pltpu.VMEM((SLICE,), jnp.float32),
            pltpu.VMEM((SLICE,), jnp.float32),
            pltpu.VMEM_SHARED((NPAD, FH), jnp.float32),
            pltpu.SemaphoreType.DMA,
            pltpu.SemaphoreType.DMA,
            pltpu.SemaphoreType.DMA,
            pltpu.SemaphoreType.DMA,
            pltpu.SemaphoreType.DMA,
            pltpu.SemaphoreType.DMA,
            pltpu.SemaphoreType.DMA,
            pltpu.SemaphoreType.DMA,
            pltpu.SemaphoreType.DMA,
            pltpu.SemaphoreType.DMA,
        ],
    )
    return kfn(a_src, a_dst, src, dst, w_e, dpart, h2)


def _post_body(s_ref, asel_ref, h2a_ref, h2b_ref, b_ref, o_ref):
    s = jnp.concatenate([s_ref[0], s_ref[1]], axis=-1)
    h = jnp.concatenate([h2a_ref[...], h2b_ref[...]], axis=-1)
    v = s + asel_ref[...] * h + b_ref[...]
    o_ref[...] = jnp.where(v > 0, v, jnp.exp(jnp.minimum(v, 0.0)) - 1.0)


def _tc_epilogue(s2, alpha_self, h2, bias):
    blk = N // 10
    return pl.pallas_call(
        _post_body,
        grid=(10,),
        in_specs=[
            pl.BlockSpec((2, blk, FH), lambda i: (0, i, 0)),
            pl.BlockSpec((blk, 1), lambda i: (i, 0)),
            pl.BlockSpec((blk, FH), lambda i: (i, 0)),
            pl.BlockSpec((blk, FH), lambda i: (10 + i, 0)),
            pl.BlockSpec((1, F), lambda i: (0, 0)),
        ],
        out_specs=pl.BlockSpec((blk, F), lambda i: (i, 0)),
        out_shape=jax.ShapeDtypeStruct((N, F), jnp.float32),
    )(s2, alpha_self, h2, h2, bias.reshape(1, F))


@jax.jit
def kernel(x, edge_index, batch, W, att_src, att_dst, bias):
    src = edge_index[0]
    dst = edge_index[1]
    h2, a_src2, a_dst2 = _tc_prologue(x, W, att_src.reshape(1, F),
                                      att_dst.reshape(1, F))
    a_src = jnp.pad(a_src2.reshape(N), (0, NPAD - N))
    a_dst = jnp.pad(a_dst2.reshape(N), (0, NPAD - N))

    w_e, dpart = _sc_denoms(a_src, a_dst, src, dst)

    alpha_all, _dn, s2 = _sc_rows(
        a_src, a_dst, src, dst, w_e, dpart, h2)

    s_halves = s2.reshape(2, NPAD, FH)[:, :N, :]
    out = _tc_epilogue(s_halves, alpha_all[E:E + N].reshape(N, 1), h2, bias)

    loop = jnp.arange(N, dtype=edge_index.dtype)
    ei = jnp.concatenate([edge_index, jnp.stack([loop, loop], axis=0)], axis=1)
    alpha = alpha_all[:E + N].reshape(E + N, 1)
    return out, ei, alpha


# speedup vs baseline: 35.1276x; 1.0129x over previous
import jax
import jax.numpy as jnp
from jax import lax
from jax.experimental import pallas as pl
from jax.experimental.pallas import tpu as pltpu
from jax.experimental.pallas import tpu_sc as plsc

N = 10000
E = 320000
F = 128
FH = F // 2
NPAD = 10240
SLICE = NPAD // 16
E_T = E // 16
E_W = E // 32
ECH = 2000
ECHB = 4000
ROWBLK = 32


def _pre_body(x_ref, w_ref, as_ref, ad_ref, h2_ref, asrc_ref, adst_ref):
    half = pl.program_id(1)
    hh = jnp.dot(x_ref[...], w_ref[0], preferred_element_type=jnp.float32)
    h2_ref[...] = hh
    pa = jnp.sum(hh * as_ref[0], axis=1, keepdims=True)
    pb = jnp.sum(hh * ad_ref[0], axis=1, keepdims=True)

    @pl.when(half == 0)
    def _():
        asrc_ref[...] = pa
        adst_ref[...] = pb

    @pl.when(half != 0)
    def _():
        asrc_ref[...] = asrc_ref[...] + pa
        adst_ref[...] = adst_ref[...] + pb


def _tc_prologue(x, W, att_src, att_dst):
    blk = N // 10
    return pl.pallas_call(
        _pre_body,
        grid=(10, 2),
        in_specs=[
            pl.BlockSpec((blk, F), lambda i, j: (i, 0)),
            pl.BlockSpec((1, F, FH), lambda i, j: (j, 0, 0)),
            pl.BlockSpec((1, 1, FH), lambda i, j: (j, 0, 0)),
            pl.BlockSpec((1, 1, FH), lambda i, j: (j, 0, 0)),
        ],
        out_specs=[
            pl.BlockSpec((blk, FH), lambda i, j: (j * 10 + i, 0)),
            pl.BlockSpec((blk, 1), lambda i, j: (i, 0)),
            pl.BlockSpec((blk, 1), lambda i, j: (i, 0)),
        ],
        out_shape=[
            jax.ShapeDtypeStruct((2 * N, FH), jnp.float32),
            jax.ShapeDtypeStruct((N, 1), jnp.float32),
            jax.ShapeDtypeStruct((N, 1), jnp.float32),
        ],
    )(x, jnp.stack([W[:, :FH], W[:, FH:]]),
      jnp.stack([att_src[:, :FH], att_src[:, FH:]]),
      jnp.stack([att_dst[:, :FH], att_dst[:, FH:]]))


def _leaky_exp(a):
    return jnp.exp(jnp.where(a > 0, a, 0.2 * a))


def _k2a_body(asrc_hbm, adst_hbm, src_hbm, dst_hbm,
              w_hbm, dpart_hbm,
              srcc0_v, srcc1_v, dstc0_v, dstc1_v,
              av0_v, av1_v, bv0_v, bv1_v, sbuf_v,
              shd_s, sema0, sema1, semb0, semb1):
    cid = lax.axis_index("c")
    sid = lax.axis_index("s")
    base = sid * SLICE
    e0 = (cid * 16 + sid) * E_W

    zero16 = jnp.zeros((16,), jnp.float32)
    srccs = (srcc0_v, srcc1_v)
    dstcs = (dstc0_v, dstc1_v)
    avs = (av0_v, av1_v)
    bvs = (bv0_v, bv1_v)
    semas = (sema0, sema1)
    sembs = (semb0, semb1)
    NCH = E_W // ECH

    def _zero_sbuf(i, _):
        sbuf_v[pl.ds(i * 16, 16)] = zero16
        return 0
    lax.fori_loop(0, SLICE // 16, _zero_sbuf, 0)
    pltpu.sync_copy(sbuf_v, shd_s.at[pl.ds(base, SLICE)])
    plsc.subcore_barrier()

    def _make_wcompute(p):
        def _wcompute(i, _):
            a16 = avs[p][pl.ds(i * 16, 16)] + bvs[p][pl.ds(i * 16, 16)]
            avs[p][pl.ds(i * 16, 16)] = _leaky_exp(a16)
            return 0
        return _wcompute

    def _prefetch(c, p):
        off = e0 + c * ECH
        pltpu.sync_copy(src_hbm.at[pl.ds(off, ECH)], srccs[p])
        pltpu.sync_copy(dst_hbm.at[pl.ds(off, ECH)], dstcs[p])
        pltpu.async_copy(asrc_hbm.at[srccs[p]], avs[p], semas[p])
        pltpu.async_copy(adst_hbm.at[dstcs[p]], bvs[p], sembs[p])

    _prefetch(0, 0)
    for c in range(NCH):
        p = c % 2
        if c + 1 < NCH:
            _prefetch(c + 1, (c + 1) % 2)
        off = e0 + c * ECH
        pltpu.make_async_copy(asrc_hbm.at[srccs[p]], avs[p], semas[p]).wait()
        pltpu.make_async_copy(adst_hbm.at[dstcs[p]], bvs[p], sembs[p]).wait()
        lax.fori_loop(0, ECH // 16, _make_wcompute(p), 0)
        pltpu.sync_copy(avs[p], w_hbm.at[pl.ds(off, ECH)])
        pltpu.sync_copy(avs[p], shd_s.at[dstcs[p]], add=True)

    plsc.subcore_barrier()
    pltpu.sync_copy(shd_s.at[pl.ds(base, SLICE)],
                    dpart_hbm.at[pl.ds(cid * NPAD + base, SLICE)])


def _sc_denoms(a_src, a_dst, src, dst):
    mesh = plsc.VectorSubcoreMesh(core_axis_name="c", subcore_axis_name="s")
    kfn = pl.kernel(
        _k2a_body,
        out_type=[
            jax.ShapeDtypeStruct((E,), jnp.float32),
            jax.ShapeDtypeStruct((2 * NPAD,), jnp.float32),
        ],
        mesh=mesh,
        compiler_params=pltpu.CompilerParams(needs_layout_passes=False),
        scratch_types=[
            pltpu.VMEM((ECH,), jnp.int32),
            pltpu.VMEM((ECH,), jnp.int32),
            pltpu.VMEM((ECH,), jnp.int32),
            pltpu.VMEM((ECH,), jnp.int32),
            pltpu.VMEM((ECH,), jnp.float32),
            pltpu.VMEM((ECH,), jnp.float32),
            pltpu.VMEM((ECH,), jnp.float32),
            pltpu.VMEM((ECH,), jnp.float32),
            pltpu.VMEM((SLICE,), jnp.float32),
            pltpu.VMEM_SHARED((NPAD,), jnp.float32),
            pltpu.SemaphoreType.DMA,
            pltpu.SemaphoreType.DMA,
            pltpu.SemaphoreType.DMA,
            pltpu.SemaphoreType.DMA,
        ],
    )
    return kfn(a_src, a_dst, src, dst)


def _k2b_body(asrc_hbm, adst_hbm, src_hbm, dst_hbm, w_hbm, dpart_hbm, h2_hbm,
              alpha_e_hbm, dntot_hbm, s2_hbm,
              srcc_v, dstc_v, wv_v, dnv_v,
              gb0_v, gb1_v, gb2_v, gb3_v, gb4_v, gb5_v,
              sb0_v, sb1_v, sb2_v,
              asl_v, bsl_v, d0_v, d1_v,
              shS_s, sem, gsem0, gsem1, gsem2, gsem3, gsem4, gsem5,
              ssem0, ssem1, ssem2):
    cid = lax.axis_index("c")
    sid = lax.axis_index("s")
    base = sid * SLICE
    e0 = sid * E_T
    crow = cid * N

    zero16 = jnp.zeros((16,), jnp.float32)
    G = ECHB // ROWBLK
    gbufs = (gb0_v, gb1_v, gb2_v, gb3_v, gb4_v, gb5_v)
    sbufs = (sb0_v, sb1_v, sb2_v)
    gsems = (gsem0, gsem1, gsem2, gsem3, gsem4, gsem5)
    ssems = (ssem0, ssem1, ssem2)

    def _zero_rows(i, _):
        gb0_v[i // 4, pl.ds((i % 4) * 16, 16)] = zero16
        return 0
    lax.fori_loop(0, ROWBLK * 4, _zero_rows, 0)
    def _zero_s(k, _):
        pltpu.sync_copy(gb0_v, shS_s.at[pl.ds(base + k * ROWBLK, ROWBLK)])
        return 0
    lax.fori_loop(0, SLICE // ROWBLK, _zero_s, 0)

    pltpu.sync_copy(asrc_hbm.at[pl.ds(base, SLICE)], asl_v)
    pltpu.sync_copy(adst_hbm.at[pl.ds(base, SLICE)], bsl_v)
    pltpu.sync_copy(dpart_hbm.at[pl.ds(base, SLICE)], d0_v)
    pltpu.sync_copy(dpart_hbm.at[pl.ds(NPAD + base, SLICE)], d1_v)
    def _selfloop(i, _):
        idx = pl.ds(i * 16, 16)
        wsel = _leaky_exp(asl_v[idx] + bsl_v[idx])
        tot = d0_v[idx] + d1_v[idx] + wsel
        d0_v[idx] = tot
        asl_v[idx] = wsel / (tot + 1e-16)
        return 0
    lax.fori_loop(0, SLICE // 16, _selfloop, 0)
    pltpu.sync_copy(d0_v, dntot_hbm.at[pl.ds(base, SLICE)])
    @pl.when(cid == 0)
    def _():
        pltpu.sync_copy(asl_v, alpha_e_hbm.at[pl.ds(E + base, SLICE)])
    plsc.subcore_barrier()

    def _alpha(i, _):
        idx = pl.ds(i * 16, 16)
        wv_v[idx] = wv_v[idx] / (dnv_v[idx] + 1e-16)
        srcc_v[idx] = srcc_v[idx] + crow
        return 0

    def _sidx(g, half):
        return srcc_v[pl.ds(g * ROWBLK + half * 16, 16)]

    def _didx(g, half):
        return dstc_v[pl.ds(g * ROWBLK + half * 16, 16)]

    def _start_g(g, b):
        for half in range(2):
            pltpu.async_copy(h2_hbm.at[_sidx(g, half)],
                             gbufs[b].at[pl.ds(half * 16, 16)], gsems[b])

    def _wait_g(g, b):
        for half in range(2):
            pltpu.make_async_copy(h2_hbm.at[_sidx(g, half)],
                                  gbufs[b].at[pl.ds(half * 16, 16)],
                                  gsems[b]).wait()

    def _start_s(g, sb):
        for half in range(2):
            pltpu.async_copy(sbufs[sb].at[pl.ds(half * 16, 16)],
                             shS_s.at[_didx(g, half)], ssems[sb], add=True)

    def _wait_s(g, sb):
        for half in range(2):
            pltpu.make_async_copy(sbufs[sb].at[pl.ds(half * 16, 16)],
                                  shS_s.at[_didx(g, half)], ssems[sb]).wait()

    def _scale(g, b, sb):
        for half in range(2):
            an = wv_v[pl.ds(g * ROWBLK + half * 16, 16)]
            for l in range(16):
                a_l = an[l]
                r = half * 16 + l
                for j in range(4):
                    sbufs[sb][r, pl.ds(j * 16, 16)] = (
                        gbufs[b][r, pl.ds(j * 16, 16)] * a_l)
        return 0

    def _chunk(c, _):
        off = e0 + c * ECHB
        pltpu.async_copy(src_hbm.at[pl.ds(off, ECHB)], srcc_v, gsem0)
        pltpu.async_copy(dst_hbm.at[pl.ds(off, ECHB)], dstc_v, gsem1)
        pltpu.async_copy(w_hbm.at[pl.ds(off, ECHB)], wv_v, gsem2)
        pltpu.make_async_copy(dst_hbm.at[pl.ds(off, ECHB)], dstc_v, gsem1).wait()
        dn_cp = pltpu.async_copy(dntot_hbm.at[dstc_v], dnv_v, sem)
        pltpu.make_async_copy(src_hbm.at[pl.ds(off, ECHB)], srcc_v, gsem0).wait()
        pltpu.make_async_copy(w_hbm.at[pl.ds(off, ECHB)], wv_v, gsem2).wait()
        dn_cp.wait()
        lax.fori_loop(0, ECHB // 16, _alpha, 0)
        @pl.when(cid == 0)
        def _():
            pltpu.sync_copy(wv_v, alpha_e_hbm.at[pl.ds(off, ECHB)])

        for b in range(6):
            _start_g(b, b)
        for b in range(6):
            _wait_g(b, b)
            if b >= 3:
                _wait_s(b - 3, b % 3)
            _scale(b, b, b % 3)
            _start_g(b + 6, b)
            _start_s(b, b % 3)

        def _pipe(k, _):
            for b in range(6):
                g = 6 * k + b
                _wait_g(g, b)
                _wait_s(g - 3, b % 3)
                _scale(g, b, b % 3)
                @pl.when(g + 6 < G)
                def _():
                    _start_g(g + 6, b)
                _start_s(g, b % 3)
            return 0
        lax.fori_loop(1, G // 6, _pipe, 0)

        for g in range(6 * (G // 6), G):
            b = g % 6
            _wait_g(g, b)
            _wait_s(g - 3, b % 3)
            _scale(g, b, b % 3)
            _start_s(g, b % 3)
        for g in range(G - 3, G):
            _wait_s(g, g % 3)
        return 0

    lax.fori_loop(0, E_T // ECHB, _chunk, 0)

    plsc.subcore_barrier()
    pltpu.sync_copy(shS_s.at[pl.ds(base, SLICE)],
                    s2_hbm.at[pl.ds(cid * NPAD + base, SLICE)])


def _sc_rows(a_src, a_dst, src, dst, w_e, dpart, h2):
    mesh = plsc.VectorSubcoreMesh(core_axis_name="c", subcore_axis_name="s")
    kfn = pl.kernel(
        _k2b_body,
        out_type=[
            jax.ShapeDtypeStruct((E + NPAD,), jnp.float32),
            jax.ShapeDtypeStruct((NPAD,), jnp.float32),
            jax.ShapeDtypeStruct((2 * NPAD, FH), jnp.float32),
        ],
        mesh=mesh,
        compiler_params=pltpu.CompilerParams(needs_layout_passes=False,
                                             use_tc_tiling_on_sc=False),
        scratch_types=[
            pltpu.VMEM((ECHB,), jnp.int32),
            pltpu.VMEM((ECHB,), jnp.int32),
            pltpu.VMEM((ECHB,), jnp.float32),
            pltpu.VMEM((ECHB,), jnp.float32),
            pltpu.VMEM((ROWBLK, FH), jnp.float32),
            pltpu.VMEM((ROWBLK, FH), jnp.float32),
            pltpu.VMEM((ROWBLK, FH), jnp.float32),
            pltpu.VMEM((ROWBLK, FH), jnp.float32),
            pltpu.VMEM((ROWBLK, FH), jnp.float32),
            pltpu.VMEM((ROWBLK, FH), jnp.float32),
            pltpu.VMEM((ROWBLK, FH), jnp.float32),
            pltpu.VMEM((ROWBLK, FH), jnp.float32),
            pltpu.VMEM((ROWBLK, FH), jnp.float32),
            pltpu.VMEM((SLICE,), jnp.float32),
            pltpu.VMEM((SLICE,), jnp.float32),
            pltpu.VMEM((SLICE,), jnp.float32),
            pltpu.VMEM((SLICE,), jnp.float32),
            pltpu.VMEM_SHARED((NPAD, FH), jnp.float32),
            pltpu.SemaphoreType.DMA,
            pltpu.SemaphoreType.DMA,
            pltpu.SemaphoreType.DMA,
            pltpu.SemaphoreType.DMA,
            pltpu.SemaphoreType.DMA,
            pltpu.SemaphoreType.DMA,
            pltpu.SemaphoreType.DMA,
            pltpu.SemaphoreType.DMA,
            pltpu.SemaphoreType.DMA,
            pltpu.SemaphoreType.DMA,
        ],
    )
    return kfn(a_src, a_dst, src, dst, w_e, dpart, h2)


def _post_body(s_ref, asel_ref, h2a_ref, h2b_ref, b_ref, o_ref):
    s = jnp.concatenate([s_ref[0], s_ref[1]], axis=-1)
    h = jnp.concatenate([h2a_ref[...], h2b_ref[...]], axis=-1)
    v = s + asel_ref[...] * h + b_ref[...]
    o_ref[...] = jnp.where(v > 0, v, jnp.exp(jnp.minimum(v, 0.0)) - 1.0)


def _tc_epilogue(s2, alpha_self, h2, bias):
    blk = N // 10
    return pl.pallas_call(
        _post_body,
        grid=(10,),
        in_specs=[
            pl.BlockSpec((2, blk, FH), lambda i: (0, i, 0)),
            pl.BlockSpec((blk, 1), lambda i: (i, 0)),
            pl.BlockSpec((blk, FH), lambda i: (i, 0)),
            pl.BlockSpec((blk, FH), lambda i: (10 + i, 0)),
            pl.BlockSpec((1, F), lambda i: (0, 0)),
        ],
        out_specs=pl.BlockSpec((blk, F), lambda i: (i, 0)),
        out_shape=jax.ShapeDtypeStruct((N, F), jnp.float32),
    )(s2, alpha_self, h2, h2, bias.reshape(1, F))


@jax.jit
def kernel(x, edge_index, batch, W, att_src, att_dst, bias):
    src = edge_index[0]
    dst = edge_index[1]
    h2, a_src2, a_dst2 = _tc_prologue(x, W, att_src.reshape(1, F),
                                      att_dst.reshape(1, F))
    a_src = jnp.pad(a_src2.reshape(N), (0, NPAD - N))
    a_dst = jnp.pad(a_dst2.reshape(N), (0, NPAD - N))

    w_e, dpart = _sc_denoms(a_src, a_dst, src, dst)

    alpha_all, _dn, s2 = _sc_rows(
        a_src, a_dst, src, dst, w_e, dpart, h2)

    s_halves = s2.reshape(2, NPAD, FH)[:, :N, :]
    out = _tc_epilogue(s_halves, alpha_all[E:E + N].reshape(N, 1), h2, bias)

    loop = jnp.arange(N, dtype=edge_index.dtype)
    ei = jnp.concatenate([edge_index, jnp.stack([loop, loop], axis=0)], axis=1)
    alpha = alpha_all[:E + N].reshape(E + N, 1)
    return out, ei, alpha


# speedup vs baseline: 35.1285x; 1.0000x over previous
import jax
import jax.numpy as jnp
from jax import lax
from jax.experimental import pallas as pl
from jax.experimental.pallas import tpu as pltpu
from jax.experimental.pallas import tpu_sc as plsc

N = 10000
E = 320000
F = 128
FH = F // 2
NPAD = 10240
SLICE = NPAD // 16
E_T = E // 16
E_W = E // 32
ECH = 2000
ECHB = 4000
ROWBLK = 32


def _pre_body(x_ref, w_ref, as_ref, ad_ref, h2_ref, asrc_ref, adst_ref):
    half = pl.program_id(1)
    hh = jnp.dot(x_ref[...], w_ref[0], preferred_element_type=jnp.float32)
    h2_ref[...] = hh
    pa = jnp.sum(hh * as_ref[0], axis=1, keepdims=True)
    pb = jnp.sum(hh * ad_ref[0], axis=1, keepdims=True)

    @pl.when(half == 0)
    def _():
        asrc_ref[...] = pa
        adst_ref[...] = pb

    @pl.when(half != 0)
    def _():
        asrc_ref[...] = asrc_ref[...] + pa
        adst_ref[...] = adst_ref[...] + pb


def _tc_prologue(x, W, att_src, att_dst):
    blk = N // 10
    return pl.pallas_call(
        _pre_body,
        grid=(10, 2),
        in_specs=[
            pl.BlockSpec((blk, F), lambda i, j: (i, 0)),
            pl.BlockSpec((1, F, FH), lambda i, j: (j, 0, 0)),
            pl.BlockSpec((1, 1, FH), lambda i, j: (j, 0, 0)),
            pl.BlockSpec((1, 1, FH), lambda i, j: (j, 0, 0)),
        ],
        out_specs=[
            pl.BlockSpec((blk, FH), lambda i, j: (j * 10 + i, 0)),
            pl.BlockSpec((blk, 1), lambda i, j: (i, 0)),
            pl.BlockSpec((blk, 1), lambda i, j: (i, 0)),
        ],
        out_shape=[
            jax.ShapeDtypeStruct((2 * N, FH), jnp.float32),
            jax.ShapeDtypeStruct((N, 1), jnp.float32),
            jax.ShapeDtypeStruct((N, 1), jnp.float32),
        ],
    )(x, jnp.stack([W[:, :FH], W[:, FH:]]),
      jnp.stack([att_src[:, :FH], att_src[:, FH:]]),
      jnp.stack([att_dst[:, :FH], att_dst[:, FH:]]))


def _leaky_exp(a):
    return jnp.exp(jnp.where(a > 0, a, 0.2 * a))


def _k2a_body(asrc_hbm, adst_hbm, src_hbm, dst_hbm,
              w_hbm, dpart_hbm,
              srcc0_v, srcc1_v, dstc0_v, dstc1_v,
              av0_v, av1_v, bv0_v, bv1_v, sbuf_v,
              shd_s, sema0, sema1, semb0, semb1):
    cid = lax.axis_index("c")
    sid = lax.axis_index("s")
    base = sid * SLICE
    e0 = (cid * 16 + sid) * E_W

    zero16 = jnp.zeros((16,), jnp.float32)
    srccs = (srcc0_v, srcc1_v)
    dstcs = (dstc0_v, dstc1_v)
    avs = (av0_v, av1_v)
    bvs = (bv0_v, bv1_v)
    semas = (sema0, sema1)
    sembs = (semb0, semb1)
    NCH = E_W // ECH

    def _zero_sbuf(i, _):
        sbuf_v[pl.ds(i * 16, 16)] = zero16
        return 0
    lax.fori_loop(0, SLICE // 16, _zero_sbuf, 0)
    pltpu.sync_copy(sbuf_v, shd_s.at[pl.ds(base, SLICE)])
    plsc.subcore_barrier()

    def _make_wcompute(p):
        def _wcompute(i, _):
            a16 = avs[p][pl.ds(i * 16, 16)] + bvs[p][pl.ds(i * 16, 16)]
            avs[p][pl.ds(i * 16, 16)] = _leaky_exp(a16)
            return 0
        return _wcompute

    def _prefetch(c, p):
        off = e0 + c * ECH
        pltpu.async_copy(src_hbm.at[pl.ds(off, ECH)], srccs[p], semas[p])
        pltpu.async_copy(dst_hbm.at[pl.ds(off, ECH)], dstcs[p], sembs[p])
        pltpu.make_async_copy(src_hbm.at[pl.ds(off, ECH)], srccs[p],
                              semas[p]).wait()
        pltpu.make_async_copy(dst_hbm.at[pl.ds(off, ECH)], dstcs[p],
                              sembs[p]).wait()
        pltpu.async_copy(asrc_hbm.at[srccs[p]], avs[p], semas[p])
        pltpu.async_copy(adst_hbm.at[dstcs[p]], bvs[p], sembs[p])

    _prefetch(0, 0)
    for c in range(NCH):
        p = c % 2
        if c + 1 < NCH:
            _prefetch(c + 1, (c + 1) % 2)
        off = e0 + c * ECH
        pltpu.make_async_copy(asrc_hbm.at[srccs[p]], avs[p], semas[p]).wait()
        pltpu.make_async_copy(adst_hbm.at[dstcs[p]], bvs[p], sembs[p]).wait()
        lax.fori_loop(0, ECH // 16, _make_wcompute(p), 0)
        pltpu.sync_copy(avs[p], w_hbm.at[pl.ds(off, ECH)])
        pltpu.sync_copy(avs[p], shd_s.at[dstcs[p]], add=True)

    plsc.subcore_barrier()
    pltpu.sync_copy(shd_s.at[pl.ds(base, SLICE)],
                    dpart_hbm.at[pl.ds(cid * NPAD + base, SLICE)])


def _sc_denoms(a_src, a_dst, src, dst):
    mesh = plsc.VectorSubcoreMesh(core_axis_name="c", subcore_axis_name="s")
    kfn = pl.kernel(
        _k2a_body,
        out_type=[
            jax.ShapeDtypeStruct((E,), jnp.float32),
            jax.ShapeDtypeStruct((2 * NPAD,), jnp.float32),
        ],
        mesh=mesh,
        compiler_params=pltpu.CompilerParams(needs_layout_passes=False),
        scratch_types=[
            pltpu.VMEM((ECH,), jnp.int32),
            pltpu.VMEM((ECH,), jnp.int32),
            pltpu.VMEM((ECH,), jnp.int32),
            pltpu.VMEM((ECH,), jnp.int32),
            pltpu.VMEM((ECH,), jnp.float32),
            pltpu.VMEM((ECH,), jnp.float32),
            pltpu.VMEM((ECH,), jnp.float32),
            pltpu.VMEM((ECH,), jnp.float32),
            pltpu.VMEM((SLICE,), jnp.float32),
            pltpu.VMEM_SHARED((NPAD,), jnp.float32),
            pltpu.SemaphoreType.DMA,
            pltpu.SemaphoreType.DMA,
            pltpu.SemaphoreType.DMA,
            pltpu.SemaphoreType.DMA,
        ],
    )
    return kfn(a_src, a_dst, src, dst)


def _k2b_body(asrc_hbm, adst_hbm, src_hbm, dst_hbm, w_hbm, dpart_hbm, h2_hbm,
              alpha_e_hbm, dntot_hbm, s2_hbm,
              srcc_v, dstc_v, wv_v, dnv_v,
              gb0_v, gb1_v, gb2_v, gb3_v, gb4_v, gb5_v,
              sb0_v, sb1_v, sb2_v,
              asl_v, bsl_v, d0_v, d1_v,
              shS_s, sem, gsem0, gsem1, gsem2, gsem3, gsem4, gsem5,
              ssem0, ssem1, ssem2):
    cid = lax.axis_index("c")
    sid = lax.axis_index("s")
    base = sid * SLICE
    e0 = sid * E_T
    crow = cid * N

    zero16 = jnp.zeros((16,), jnp.float32)
    G = ECHB // ROWBLK
    gbufs = (gb0_v, gb1_v, gb2_v, gb3_v, gb4_v, gb5_v)
    sbufs = (sb0_v, sb1_v, sb2_v)
    gsems = (gsem0, gsem1, gsem2, gsem3, gsem4, gsem5)
    ssems = (ssem0, ssem1, ssem2)

    def _zero_rows(i, _):
        gb0_v[i // 4, pl.ds((i % 4) * 16, 16)] = zero16
        return 0
    lax.fori_loop(0, ROWBLK * 4, _zero_rows, 0)
    def _zero_s(k, _):
        pltpu.sync_copy(gb0_v, shS_s.at[pl.ds(base + k * ROWBLK, ROWBLK)])
        return 0
    lax.fori_loop(0, SLICE // ROWBLK, _zero_s, 0)

    pltpu.sync_copy(asrc_hbm.at[pl.ds(base, SLICE)], asl_v)
    pltpu.sync_copy(adst_hbm.at[pl.ds(base, SLICE)], bsl_v)
    pltpu.sync_copy(dpart_hbm.at[pl.ds(base, SLICE)], d0_v)
    pltpu.sync_copy(dpart_hbm.at[pl.ds(NPAD + base, SLICE)], d1_v)
    def _selfloop(i, _):
        idx = pl.ds(i * 16, 16)
        wsel = _leaky_exp(asl_v[idx] + bsl_v[idx])
        tot = d0_v[idx] + d1_v[idx] + wsel
        d0_v[idx] = tot
        asl_v[idx] = wsel / (tot + 1e-16)
        return 0
    lax.fori_loop(0, SLICE // 16, _selfloop, 0)
    pltpu.sync_copy(d0_v, dntot_hbm.at[pl.ds(base, SLICE)])
    @pl.when(cid == 0)
    def _():
        pltpu.sync_copy(asl_v, alpha_e_hbm.at[pl.ds(E + base, SLICE)])
    plsc.subcore_barrier()

    def _alpha(i, _):
        idx = pl.ds(i * 16, 16)
        wv_v[idx] = wv_v[idx] / (dnv_v[idx] + 1e-16)
        srcc_v[idx] = srcc_v[idx] + crow
        return 0

    def _sidx(g, half):
        return srcc_v[pl.ds(g * ROWBLK + half * 16, 16)]

    def _didx(g, half):
        return dstc_v[pl.ds(g * ROWBLK + half * 16, 16)]

    def _start_g(g, b):
        for half in range(2):
            pltpu.async_copy(h2_hbm.at[_sidx(g, half)],
                             gbufs[b].at[pl.ds(half * 16, 16)], gsems[b])

    def _wait_g(g, b):
        for half in range(2):
            pltpu.make_async_copy(h2_hbm.at[_sidx(g, half)],
                                  gbufs[b].at[pl.ds(half * 16, 16)],
                                  gsems[b]).wait()

    def _start_s(g, sb):
        for half in range(2):
            pltpu.async_copy(sbufs[sb].at[pl.ds(half * 16, 16)],
                             shS_s.at[_didx(g, half)], ssems[sb], add=True)

    def _wait_s(g, sb):
        for half in range(2):
            pltpu.make_async_copy(sbufs[sb].at[pl.ds(half * 16, 16)],
                                  shS_s.at[_didx(g, half)], ssems[sb]).wait()

    def _scale(g, b, sb):
        for half in range(2):
            an = wv_v[pl.ds(g * ROWBLK + half * 16, 16)]
            for l in range(16):
                a_l = an[l]
                r = half * 16 + l
                for j in range(4):
                    sbufs[sb][r, pl.ds(j * 16, 16)] = (
                        gbufs[b][r, pl.ds(j * 16, 16)] * a_l)
        return 0

    def _chunk(c, _):
        off = e0 + c * ECHB
        pltpu.async_copy(src_hbm.at[pl.ds(off, ECHB)], srcc_v, gsem0)
        pltpu.async_copy(dst_hbm.at[pl.ds(off, ECHB)], dstc_v, gsem1)
        pltpu.async_copy(w_hbm.at[pl.ds(off, ECHB)], wv_v, gsem2)
        pltpu.make_async_copy(dst_hbm.at[pl.ds(off, ECHB)], dstc_v, gsem1).wait()
        dn_cp = pltpu.async_copy(dntot_hbm.at[dstc_v], dnv_v, sem)
        pltpu.make_async_copy(src_hbm.at[pl.ds(off, ECHB)], srcc_v, gsem0).wait()
        pltpu.make_async_copy(w_hbm.at[pl.ds(off, ECHB)], wv_v, gsem2).wait()
        dn_cp.wait()
        lax.fori_loop(0, ECHB // 16, _alpha, 0)
        @pl.when(cid == 0)
        def _():
            pltpu.sync_copy(wv_v, alpha_e_hbm.at[pl.ds(off, ECHB)])

        for b in range(6):
            _start_g(b, b)
        for b in range(6):
            _wait_g(b, b)
            if b >= 3:
                _wait_s(b - 3, b % 3)
            _scale(b, b, b % 3)
            _start_g(b + 6, b)
            _start_s(b, b % 3)

        def _pipe(k, _):
            for b in range(6):
                g = 6 * k + b
                _wait_g(g, b)
                _wait_s(g - 3, b % 3)
                _scale(g, b, b % 3)
                @pl.when(g + 6 < G)
                def _():
                    _start_g(g + 6, b)
                _start_s(g, b % 3)
            return 0
        lax.fori_loop(1, G // 6, _pipe, 0)

        for g in range(6 * (G // 6), G):
            b = g % 6
            _wait_g(g, b)
            _wait_s(g - 3, b % 3)
            _scale(g, b, b % 3)
            _start_s(g, b % 3)
        for g in range(G - 3, G):
            _wait_s(g, g % 3)
        return 0

    lax.fori_loop(0, E_T // ECHB, _chunk, 0)

    plsc.subcore_barrier()
    pltpu.sync_copy(shS_s.at[pl.ds(base, SLICE)],
                    s2_hbm.at[pl.ds(cid * NPAD + base, SLICE)])


def _sc_rows(a_src, a_dst, src, dst, w_e, dpart, h2):
    mesh = plsc.VectorSubcoreMesh(core_axis_name="c", subcore_axis_name="s")
    kfn = pl.kernel(
        _k2b_body,
        out_type=[
            jax.ShapeDtypeStruct((E + NPAD,), jnp.float32),
            jax.ShapeDtypeStruct((NPAD,), jnp.float32),
            jax.ShapeDtypeStruct((2 * NPAD, FH), jnp.float32),
        ],
        mesh=mesh,
        compiler_params=pltpu.CompilerParams(needs_layout_passes=False,
                                             use_tc_tiling_on_sc=False),
        scratch_types=[
            pltpu.VMEM((ECHB,), jnp.int32),
            pltpu.VMEM((ECHB,), jnp.int32),
            pltpu.VMEM((ECHB,), jnp.float32),
            pltpu.VMEM((ECHB,), jnp.float32),
            pltpu.VMEM((ROWBLK, FH), jnp.float32),
            pltpu.VMEM((ROWBLK, FH), jnp.float32),
            pltpu.VMEM((ROWBLK, FH), jnp.float32),
            pltpu.VMEM((ROWBLK, FH), jnp.float32),
            pltpu.VMEM((ROWBLK, FH), jnp.float32),
            pltpu.VMEM((ROWBLK, FH), jnp.float32),
            pltpu.VMEM((ROWBLK, FH), jnp.float32),
            pltpu.VMEM((ROWBLK, FH), jnp.float32),
            pltpu.VMEM((ROWBLK, FH), jnp.float32),
            pltpu.VMEM((SLICE,), jnp.float32),
            pltpu.VMEM((SLICE,), jnp.float32),
            pltpu.VMEM((SLICE,), jnp.float32),
            pltpu.VMEM((SLICE,), jnp.float32),
            pltpu.VMEM_SHARED((NPAD, FH), jnp.float32),
            pltpu.SemaphoreType.DMA,
            pltpu.SemaphoreType.DMA,
            pltpu.SemaphoreType.DMA,
            pltpu.SemaphoreType.DMA,
            pltpu.SemaphoreType.DMA,
            pltpu.SemaphoreType.DMA,
            pltpu.SemaphoreType.DMA,
            pltpu.SemaphoreType.DMA,
            pltpu.SemaphoreType.DMA,
            pltpu.SemaphoreType.DMA,
        ],
    )
    return kfn(a_src, a_dst, src, dst, w_e, dpart, h2)


def _post_body(s_ref, asel_ref, h2a_ref, h2b_ref, b_ref, o_ref):
    s = jnp.concatenate([s_ref[0], s_ref[1]], axis=-1)
    h = jnp.concatenate([h2a_ref[...], h2b_ref[...]], axis=-1)
    v = s + asel_ref[...] * h + b_ref[...]
    o_ref[...] = jnp.where(v > 0, v, jnp.exp(jnp.minimum(v, 0.0)) - 1.0)


def _tc_epilogue(s2, alpha_self, h2, bias):
    blk = N // 10
    return pl.pallas_call(
        _post_body,
        grid=(10,),
        in_specs=[
            pl.BlockSpec((2, blk, FH), lambda i: (0, i, 0)),
            pl.BlockSpec((blk, 1), lambda i: (i, 0)),
            pl.BlockSpec((blk, FH), lambda i: (i, 0)),
            pl.BlockSpec((blk, FH), lambda i: (10 + i, 0)),
            pl.BlockSpec((1, F), lambda i: (0, 0)),
        ],
        out_specs=pl.BlockSpec((blk, F), lambda i: (i, 0)),
        out_shape=jax.ShapeDtypeStruct((N, F), jnp.float32),
    )(s2, alpha_self, h2, h2, bias.reshape(1, F))


@jax.jit
def kernel(x, edge_index, batch, W, att_src, att_dst, bias):
    src = edge_index[0]
    dst = edge_index[1]
    h2, a_src2, a_dst2 = _tc_prologue(x, W, att_src.reshape(1, F),
                                      att_dst.reshape(1, F))
    a_src = jnp.pad(a_src2.reshape(N), (0, NPAD - N))
    a_dst = jnp.pad(a_dst2.reshape(N), (0, NPAD - N))

    w_e, dpart = _sc_denoms(a_src, a_dst, src, dst)

    alpha_all, _dn, s2 = _sc_rows(
        a_src, a_dst, src, dst, w_e, dpart, h2)

    s_halves = s2.reshape(2, NPAD, FH)[:, :N, :]
    out = _tc_epilogue(s_halves, alpha_all[E:E + N].reshape(N, 1), h2, bias)

    loop = jnp.arange(N, dtype=edge_index.dtype)
    ei = jnp.concatenate([edge_index, jnp.stack([loop, loop], axis=0)], axis=1)
    alpha = alpha_all[:E + N].reshape(E + N, 1)
    return out, ei, alpha
